# baseline (speedup 1.0000x reference)
import functools

import jax
import jax.numpy as jnp
from jax import lax
from jax.experimental import pallas as pl
from jax.experimental.pallas import tpu as pltpu
from jax.experimental.pallas import tpu_sc as plsc

_N_GENE = 20000
_N_MESH = 10000
_H = 128
_NTG = 20096
_NTM = 10112
_NPG = 20224
_NPM = 10112
_EC_QUANT = 32 * 256

_MESH = plsc.VectorSubcoreMesh(core_axis_name="c", subcore_axis_name="s")
_F32 = jnp.float32
_I32 = jnp.int32


def _wid():
    return lax.axis_index("s") * 2 + lax.axis_index("c")


def _mm_stats_call(x, w, nt):
    n, kp = x.shape
    nb = nt // 128
    tail = n - (nb - 1) * 128

    def body(x_ref, w_ref, h_ref, st_ref, acc):
        i = pl.program_id(0)

        @pl.when(i == 0)
        def _():
            acc[...] = jnp.zeros_like(acc)

        h = jnp.dot(x_ref[...], w_ref[...], preferred_element_type=_F32)

        @pl.when(i == nb - 1)
        def _():
            rows = lax.broadcasted_iota(_I32, (128, _H), 0)
            h_ref[...] = jnp.where(rows < tail, h, 0.0)
            acc[0:1, :] += jnp.sum(h_ref[...], axis=0, keepdims=True)
            acc[1:2, :] += jnp.sum(h_ref[...] * h_ref[...], axis=0,
                                   keepdims=True)
            st_ref[...] = acc[...]

        @pl.when(i < nb - 1)
        def _():
            h_ref[...] = h
            acc[0:1, :] += jnp.sum(h, axis=0, keepdims=True)
            acc[1:2, :] += jnp.sum(h * h, axis=0, keepdims=True)

    return pl.pallas_call(
        body,
        grid=(nb,),
        in_specs=[
            pl.BlockSpec((128, kp), lambda i: (i, 0)),
            pl.BlockSpec((kp, _H), lambda i: (0, 0)),
        ],
        out_specs=[
            pl.BlockSpec((128, _H), lambda i: (i, 0)),
            pl.BlockSpec((8, _H), lambda i: (0, 0)),
        ],
        out_shape=[
            jax.ShapeDtypeStruct((nt, _H), _F32),
            jax.ShapeDtypeStruct((8, _H), _F32),
        ],
        scratch_shapes=[pltpu.VMEM((8, _H), _F32)],
    )(x, w)


def _bn_apply_call(h, stats, gamma, beta, n_true):
    nt = h.shape[0]
    nb = nt // 128
    inv_n = 1.0 / float(n_true)

    def body(h_ref, st_ref, g_ref, b_ref, y_ref):
        mu = st_ref[0:1, :] * inv_n
        var = st_ref[1:2, :] * inv_n - mu * mu
        scale = g_ref[...] * lax.rsqrt(var + 1e-5)
        y = (h_ref[...] - mu) * scale + b_ref[...]
        y_ref[...] = jnp.maximum(y, 0.0)

    return pl.pallas_call(
        body,
        grid=(nb,),
        in_specs=[
            pl.BlockSpec((128, _H), lambda i: (i, 0)),
            pl.BlockSpec((8, _H), lambda i: (0, 0)),
            pl.BlockSpec((1, _H), lambda i: (0, 0)),
            pl.BlockSpec((1, _H), lambda i: (0, 0)),
        ],
        out_specs=pl.BlockSpec((128, _H), lambda i: (i, 0)),
        out_shape=jax.ShapeDtypeStruct((nt, _H), _F32),
    )(h, stats, gamma, beta)


def _h_alpha_call(y, w, a_col, want_h):
    nt = y.shape[0]
    nb = nt // 128

    def body(*refs):
        if want_h:
            y_ref, w_ref, a_ref, h_ref, al_ref, m_ref, wv, mx = refs
        else:
            y_ref, w_ref, a_ref, al_ref, m_ref, wv, mx = refs
        i = pl.program_id(0)

        @pl.when(i == 0)
        def _():
            wv[...] = jnp.dot(w_ref[...], a_ref[...],
                              preferred_element_type=_F32)
            mx[0, 0] = -3.0e38

        yb = y_ref[...]
        if want_h:
            h_ref[...] = jnp.dot(yb, w_ref[...], preferred_element_type=_F32)
        av = jnp.dot(yb, wv[...], preferred_element_type=_F32)
        al_ref[...] = av
        mx[0, 0] = jnp.maximum(mx[0, 0], jnp.max(av))
        m_ref[...] = jnp.full((1, _H), mx[0, 0], _F32)

    out_specs = [
        pl.BlockSpec((128, 1), lambda i: (i, 0)),
        pl.BlockSpec((1, _H), lambda i: (0, 0)),
    ]
    out_shape = [
        jax.ShapeDtypeStruct((nt, 1), _F32),
        jax.ShapeDtypeStruct((1, _H), _F32),
    ]
    if want_h:
        out_specs.insert(0, pl.BlockSpec((128, _H), lambda i: (i, 0)))
        out_shape.insert(0, jax.ShapeDtypeStruct((nt, _H), _F32))

    return pl.pallas_call(
        body,
        grid=(nb,),
        in_specs=[
            pl.BlockSpec((128, _H), lambda i: (i, 0)),
            pl.BlockSpec((_H, _H), lambda i: (0, 0)),
            pl.BlockSpec((_H, 1), lambda i: (0, 0)),
        ],
        out_specs=out_specs,
        out_shape=out_shape,
        scratch_shapes=[pltpu.VMEM((_H, 1), _F32), pltpu.SMEM((1, 1), _F32)],
    )(y, w, a_col)


def _h_call(y, w):
    nt = y.shape[0]
    nb = nt // 128

    def body(y_ref, w_ref, h_ref):
        h_ref[...] = jnp.dot(y_ref[...], w_ref[...],
                             preferred_element_type=_F32)

    return pl.pallas_call(
        body,
        grid=(nb,),
        in_specs=[
            pl.BlockSpec((128, _H), lambda i: (i, 0)),
            pl.BlockSpec((_H, _H), lambda i: (0, 0)),
        ],
        out_specs=pl.BlockSpec((128, _H), lambda i: (i, 0)),
        out_shape=jax.ShapeDtypeStruct((nt, _H), _F32),
    )(y, w)


def _finalize_call(num, den, b, nt, gat):
    nb = nt // 128

    in_specs = [pl.BlockSpec((128, _H), lambda i: (i, 0))]
    args = [num]
    if gat:
        in_specs.append(pl.BlockSpec((32, 128), lambda i: (0, i)))
        args.append(den)
    in_specs.append(pl.BlockSpec((1, _H), lambda i: (0, 0)))
    args.append(b)

    def body(*refs):
        refs = list(refs)
        y_ref = refs.pop()
        b_ref = refs.pop()
        if gat:
            den_ref = refs.pop()
        big = refs[0][...]
        if gat:
            dcol = lax.dot_general(den_ref[...], jnp.ones((32, 1), _F32),
                                   (((0,), (0,)), ((), ())),
                                   preferred_element_type=_F32)
            big = big / (dcol + 1e-16)
        y_ref[...] = jnp.maximum(big + b_ref[...], 0.0)

    return pl.pallas_call(
        body,
        grid=(nb,),
        in_specs=in_specs,
        out_specs=pl.BlockSpec((128, _H), lambda i: (i, 0)),
        out_shape=jax.ShapeDtypeStruct((nt, _H), _F32),
    )(*args)


def _dinv_call(deg_parts):
    npad = deg_parts.shape[1]
    nb = npad // 128

    def body(d_ref, o_ref):
        deg = lax.dot_general(d_ref[...], jnp.ones((32, 1), _F32),
                              (((0,), (0,)), ((), ())),
                              preferred_element_type=_F32)
        o_ref[...] = jnp.where(deg > 0.0, lax.rsqrt(jnp.maximum(deg, 1e-12)),
                               0.0)

    return pl.pallas_call(
        body,
        grid=(nb,),
        in_specs=[pl.BlockSpec((32, 128), lambda i: (0, i))],
        out_specs=pl.BlockSpec((128, 1), lambda i: (i, 0)),
        out_shape=jax.ShapeDtypeStruct((npad, 1), _F32),
    )(deg_parts)


def _zero_1d(ref, nwords):
    z = jnp.zeros((16,), _F32)

    def bd(i, c):
        ref[pl.ds(i * 16, 16)] = z
        return c
    lax.fori_loop(0, nwords // 16, bd, 0)


def _deg_call(dst, npad):
    epad = dst.shape[0]
    ec = epad // 32

    @functools.partial(
        pl.kernel, mesh=_MESH,
        compiler_params=pltpu.CompilerParams(needs_layout_passes=False, use_tc_tiling_on_sc=False),
        out_type=jax.ShapeDtypeStruct((32, npad), _F32),
        scratch_types=[pltpu.VMEM((npad,), _F32), pltpu.VMEM((128,), _I32)],
    )
    def k(dst_hbm, out_hbm, deg_v, idx_v):
        wid = _wid()
        base = wid * ec
        _zero_1d(deg_v, npad)
        ones16 = jnp.ones((16,), _F32)

        def chunk(c, carry):
            pltpu.sync_copy(dst_hbm.at[pl.ds(base + c * 128, 128)], idx_v)
            for j in range(8):
                d16 = idx_v[pl.ds(j * 16, 16)]
                plsc.addupdate_scatter(deg_v, [d16], ones16)
            return carry
        lax.fori_loop(0, ec // 128, chunk, 0)
        pltpu.sync_copy(deg_v, out_hbm.at[wid])

    return k(dst)


def _gat_w_call(asrc, adst, msrc, mdst, src, dst, npad):
    epad = src.shape[0]
    ec = epad // 32
    ns = asrc.shape[0]
    nd = adst.shape[0]

    @functools.partial(
        pl.kernel, mesh=_MESH,
        compiler_params=pltpu.CompilerParams(needs_layout_passes=False, use_tc_tiling_on_sc=False),
        out_type=[jax.ShapeDtypeStruct((epad,), _F32),
                  jax.ShapeDtypeStruct((32, npad), _F32)],
        scratch_types=[
            pltpu.VMEM((ns,), _F32), pltpu.VMEM((nd,), _F32),
            pltpu.VMEM((npad,), _F32),
            pltpu.VMEM((128,), _I32), pltpu.VMEM((128,), _I32),
            pltpu.VMEM((128,), _F32),
            pltpu.VMEM((128,), _F32), pltpu.VMEM((128,), _F32),
        ],
    )
    def k(asrc_hbm, adst_hbm, msrc_hbm, mdst_hbm, src_hbm, dst_hbm,
          w_hbm, den_hbm, as_v, ad_v, den_v, si_v, di_v, w_v, ms_v, md_v):
        wid = _wid()
        base = wid * ec
        pltpu.sync_copy(asrc_hbm, as_v)
        pltpu.sync_copy(adst_hbm, ad_v)
        pltpu.sync_copy(msrc_hbm, ms_v)
        pltpu.sync_copy(mdst_hbm, md_v)
        _zero_1d(den_v, npad)
        msum = ms_v[pl.ds(0, 16)] + md_v[pl.ds(0, 16)]
        mb = jnp.where(msum > 0.0, msum, 0.2 * msum)

        def chunk(c, carry):
            off = base + c * 128
            pltpu.sync_copy(src_hbm.at[pl.ds(off, 128)], si_v)
            pltpu.sync_copy(dst_hbm.at[pl.ds(off, 128)], di_v)
            for j in range(8):
                s16 = si_v[pl.ds(j * 16, 16)]
                d16 = di_v[pl.ds(j * 16, 16)]
                e = plsc.load_gather(as_v, [s16]) + plsc.load_gather(ad_v, [d16])
                e = jnp.where(e > 0.0, e, 0.2 * e)
                ex = jnp.exp(e - mb)
                w_v[pl.ds(j * 16, 16)] = ex
                plsc.addupdate_scatter(den_v, [d16], ex)
            pltpu.sync_copy(w_v, w_hbm.at[pl.ds(off, 128)])
            return carry
        lax.fori_loop(0, ec // 128, chunk, 0)
        pltpu.sync_copy(den_v, den_hbm.at[wid])

    return k(asrc, adst, msrc, mdst, src, dst)


def _gcn_w_call(dinv, src, dst):
    epad = src.shape[0]
    ec = epad // 32
    nn = dinv.shape[0]

    @functools.partial(
        pl.kernel, mesh=_MESH,
        compiler_params=pltpu.CompilerParams(needs_layout_passes=False, use_tc_tiling_on_sc=False),
        out_type=jax.ShapeDtypeStruct((epad,), _F32),
        scratch_types=[
            pltpu.VMEM((nn,), _F32),
            pltpu.VMEM((128,), _I32), pltpu.VMEM((128,), _I32),
            pltpu.VMEM((128,), _F32),
        ],
    )
    def k(dinv_hbm, src_hbm, dst_hbm, w_hbm, dv, si_v, di_v, w_v):
        wid = _wid()
        base = wid * ec
        pltpu.sync_copy(dinv_hbm, dv)

        def chunk(c, carry):
            off = base + c * 128
            pltpu.sync_copy(src_hbm.at[pl.ds(off, 128)], si_v)
            pltpu.sync_copy(dst_hbm.at[pl.ds(off, 128)], di_v)
            for j in range(8):
                s16 = si_v[pl.ds(j * 16, 16)]
                d16 = di_v[pl.ds(j * 16, 16)]
                w_v[pl.ds(j * 16, 16)] = (plsc.load_gather(dv, [s16]) *
                                          plsc.load_gather(dv, [d16]))
            pltpu.sync_copy(w_v, w_hbm.at[pl.ds(off, 128)])
            return carry
        lax.fori_loop(0, ec // 128, chunk, 0)

    return k(dinv, src, dst)


def _v2s(vref, sref, n):
    for g in range(n // 16):
        v = vref[pl.ds(g * 16, 16)]
        for j in range(16):
            sref[g * 16 + j] = v[j]


def _bin_hist_call(dst, nbp):
    epad = dst.shape[0]
    ec = epad // 32

    @functools.partial(
        pl.kernel, mesh=_MESH,
        compiler_params=pltpu.CompilerParams(needs_layout_passes=False,
                                             use_tc_tiling_on_sc=False),
        out_type=jax.ShapeDtypeStruct((32, nbp), _I32),
        scratch_types=[pltpu.VMEM((nbp,), _I32), pltpu.VMEM((128,), _I32)],
    )
    def k(dst_hbm, out_hbm, hist_v, di_v):
        wid = _wid()
        base = wid * ec
        z16 = jnp.zeros((16,), _I32)
        one16 = jnp.ones((16,), _I32)

        def zb(i, c):
            hist_v[pl.ds(i * 16, 16)] = z16
            return c
        lax.fori_loop(0, nbp // 16, zb, 0)

        def chunk(c, carry):
            pltpu.sync_copy(dst_hbm.at[pl.ds(base + c * 128, 128)], di_v)
            for j in range(8):
                b16 = lax.shift_right_logical(di_v[pl.ds(j * 16, 16)], 7)
                plsc.addupdate_scatter(hist_v, [b16], one16)
            return carry
        lax.fori_loop(0, ec // 128, chunk, 0)
        pltpu.sync_copy(hist_v, out_hbm.at[wid])

    return k(dst)


def _bin_scatter_call(src, dst, hist, nbp):
    epad = src.shape[0]
    ec = epad // 32

    @functools.partial(
        pl.kernel, mesh=_MESH,
        compiler_params=pltpu.CompilerParams(needs_layout_passes=False,
                                             use_tc_tiling_on_sc=False),
        out_type=[jax.ShapeDtypeStruct((epad,), _I32),
                  jax.ShapeDtypeStruct((epad,), _I32),
                  jax.ShapeDtypeStruct((nbp,), _I32)],
        scratch_types=[
            pltpu.VMEM((32, nbp), _I32), pltpu.VMEM((nbp,), _I32),
            pltpu.VMEM((nbp,), _I32), pltpu.VMEM((nbp,), _I32),
            pltpu.VMEM((128,), _I32), pltpu.VMEM((128,), _I32),
            pltpu.VMEM((128,), _I32),
            pltpu.SMEM((nbp,), _I32), pltpu.SMEM((128,), _I32),
        ],
    )
    def k(src_hbm, dst_hbm, hist_hbm, ps_hbm, pd_hbm, bs_hbm,
          allh_v, tot_v, bs_v, cur_v, si_v, di_v, pos_v,
          cur_s, di_s):
        wid = _wid()
        base = wid * ec
        pltpu.sync_copy(hist_hbm, allh_v)

        for gb in range(nbp // 16):
            sl = pl.ds(gb * 16, 16)
            acc = jnp.zeros((16,), _I32)
            for t in range(32):
                acc = acc + allh_v[t, sl]
            tot_v[sl] = acc

        carry_in = jnp.int32(0)
        for gb in range(nbp // 16):
            sl = pl.ds(gb * 16, 16)
            t16 = tot_v[sl]
            inc = plsc.cumsum(t16)
            bs_v[sl] = inc - t16 + carry_in
            carry_in = carry_in + inc[15]

        for gb in range(nbp // 16):
            sl = pl.ds(gb * 16, 16)

            def pr(t, acc):
                return acc + allh_v[t, sl]
            pre = lax.fori_loop(0, wid, pr, jnp.zeros((16,), _I32))
            cur_v[sl] = bs_v[sl] + pre

        _v2s(cur_v, cur_s, nbp)
        lane0 = lax.iota(_I32, 16) == 0

        @pl.when(wid == 0)
        def _():
            pltpu.sync_copy(bs_v, bs_hbm)

        def chunk(c, carry):
            off = base + c * 128
            pltpu.sync_copy(src_hbm.at[pl.ds(off, 128)], si_v)
            pltpu.sync_copy(dst_hbm.at[pl.ds(off, 128)], di_v)
            _v2s(di_v, di_s, 128)

            def ed(r, cc):
                b = lax.shift_right_logical(di_s[r], 7)
                pos = cur_s[b]
                cur_s[b] = pos + 1
                plsc.store_scatter(pos_v, [jnp.full((16,), r, _I32)],
                                   jnp.full((16,), pos, _I32), mask=lane0)
                return cc
            lax.fori_loop(0, 128, ed, 0)
            pltpu.sync_copy(si_v, ps_hbm.at[pos_v])
            pltpu.sync_copy(di_v, pd_hbm.at[pos_v])
            return carry
        lax.fori_loop(0, ec // 128, chunk, 0)

    return k(src, dst, hist)


def _seg_rows_call(feat, psrc, pdst, w, bst, npad):
    epad = psrc.shape[0]
    nbp = bst.shape[0]
    nblk = npad // 128

    @functools.partial(
        pl.kernel, mesh=_MESH,
        compiler_params=pltpu.CompilerParams(needs_layout_passes=False,
                                             use_tc_tiling_on_sc=False),
        out_type=jax.ShapeDtypeStruct((npad, _H), _F32),
        scratch_types=[
            pltpu.VMEM((nbp,), _I32),
            pltpu.VMEM((128,), _I32), pltpu.VMEM((128,), _I32),
            pltpu.VMEM((128,), _F32),
            pltpu.VMEM((128, _H), _F32), pltpu.VMEM((128, _H), _F32),
            pltpu.SMEM((nbp,), _I32),
            pltpu.SemaphoreType.DMA,
        ],
    )
    def k(feat_hbm, ps_hbm, pd_hbm, w_hbm, bst_hbm, out_hbm,
          bs_v, si_v, di_v, w_v, rows_v, acc_v, bs_s, gsem):
        wid = _wid()
        pltpu.sync_copy(bst_hbm, bs_v)
        _v2s(bs_v, bs_s, nbp)
        z16 = jnp.zeros((16,), _F32)
        nbt = (nblk - wid + 31) // 32

        def bucket(kk, carry):
            b = wid + kk * 32
            s = bs_s[b]
            e = bs_s[b + 1]

            def za(i, c):
                for j in range(8):
                    acc_v[i, pl.ds(j * 16, 16)] = z16
                return c
            lax.fori_loop(0, 128, za, 0)

            s_al = (s // 128) * 128
            nwin = (e - s_al + 127) // 128

            iota16 = lax.iota(_I32, 16)

            def win(wn, c2):
                off = s_al + wn * 128
                pltpu.sync_copy(ps_hbm.at[pl.ds(off, 128)], si_v)
                pltpu.sync_copy(pd_hbm.at[pl.ds(off, 128)], di_v)
                pltpu.sync_copy(w_hbm.at[pl.ds(off, 128)], w_v)
                pltpu.async_copy(feat_hbm.at[si_v], rows_v, gsem).wait()

                ls, ws, vs = [], [], []
                for j in range(8):
                    sl = pl.ds(j * 16, 16)
                    pos16 = iota16 + (off + j * 16)
                    valid = jnp.logical_and(pos16 >= s, pos16 < e)
                    ls.append(di_v[sl] - b * 128)
                    ws.append(w_v[sl])
                    vs.append(valid)

                def col(c, carry):
                    lsc, wsc, vsc = carry
                    cf = jnp.full((16,), c, _I32)
                    for j in range(8):
                        r16 = iota16 + j * 16
                        v = plsc.load_gather(rows_v, [r16, cf])
                        plsc.addupdate_scatter(acc_v, [lsc[j], cf],
                                               v * wsc[j], mask=vsc[j])
                    return carry
                lax.fori_loop(0, 128, col, (tuple(ls), tuple(ws), tuple(vs)))
                return c2
            lax.fori_loop(0, nwin, win, 0)
            pltpu.sync_copy(acc_v, out_hbm.at[pl.ds(b * 128, 128)])
            return carry
        lax.fori_loop(0, nbt, bucket, 0)

    return k(feat, psrc, pdst, w, bst)


def _pad_rows(x, nt):
    return jnp.pad(x, ((0, nt - x.shape[0]), (0, 0)))


def _pad_edges(ei, dump):
    e = ei.shape[1]
    epad = -(-e // _EC_QUANT) * _EC_QUANT
    src = jnp.pad(ei[0], (0, epad - e))
    dst = jnp.pad(ei[1], (0, epad - e), constant_values=dump)
    return src, dst


def kernel(x_gene, x_mesh, params, ei_gg, ei_mm, ei_gm, ei_mg,
           edge_label_index):
    p = params
    del edge_label_index

    hg, stg = _mm_stats_call(x_gene, p["lin_g_W"], _NTG)
    g = _bn_apply_call(hg, stg, p["bn_g_g"].reshape(1, _H),
                       p["bn_g_b"].reshape(1, _H), _N_GENE)
    hm, stm = _mm_stats_call(x_mesh, p["lin_m_W"], _NTM)
    m = _bn_apply_call(hm, stm, p["bn_m_g"].reshape(1, _H),
                       p["bn_m_b"].reshape(1, _H), _N_MESH)

    nbp_g = _NPG // 128 + 2
    nbp_m = _NPM // 128 + 1
    s_gg, d_gg = _pad_edges(ei_gg, _NTG - 1)
    s_mm, d_mm = _pad_edges(ei_mm, _NTM - 1)
    s_gm, d_gm = _pad_edges(ei_gm, _NTM - 1)
    s_mg, d_mg = _pad_edges(ei_mg, _NTG - 1)
    s_gg, d_gg, bst_gg = _bin_scatter_call(
        s_gg, d_gg, _bin_hist_call(d_gg, nbp_g), nbp_g)
    s_mm, d_mm, bst_mm = _bin_scatter_call(
        s_mm, d_mm, _bin_hist_call(d_mm, nbp_m), nbp_m)
    s_gm, d_gm, bst_gm = _bin_scatter_call(
        s_gm, d_gm, _bin_hist_call(d_gm, nbp_m), nbp_m)
    s_mg, d_mg, bst_mg = _bin_scatter_call(
        s_mg, d_mg, _bin_hist_call(d_mg, nbp_g), nbp_g)

    deg_gg = _deg_call(d_gg, _NPG)
    deg_mm = _deg_call(d_mm, _NPM)
    dinv_gg = _dinv_call(deg_gg)[: _NTG, 0]
    dinv_mm = _dinv_call(deg_mm)[: _NTM, 0]
    w_gg = _gcn_w_call(dinv_gg, s_gg, d_gg)
    w_mm = _gcn_w_call(dinv_mm, s_mm, d_mm)

    for l in range(2):
        hs_g, al_g, mx_g = _h_alpha_call(
            g, p[f"gat{l}_gm_Ws"], p[f"gat{l}_gm_as"].reshape(_H, 1), True)
        al_md, mx_md = _h_alpha_call(
            m, p[f"gat{l}_gm_Wd"], p[f"gat{l}_gm_ad"].reshape(_H, 1), False)
        w_e, den = _gat_w_call(al_g[:, 0], al_md[:, 0], mx_g[0], mx_md[0],
                               s_gm, d_gm, _NPM)
        num = _seg_rows_call(hs_g, s_gm, d_gm, w_e, bst_gm, _NPM)
        nm = _finalize_call(num, den, p[f"gat{l}_gm_b"].reshape(1, _H),
                            _NTM, True)

        hs_m, al_m, mx_m = _h_alpha_call(
            m, p[f"gat{l}_mg_Ws"], p[f"gat{l}_mg_as"].reshape(_H, 1), True)
        al_gd, mx_gd = _h_alpha_call(
            g, p[f"gat{l}_mg_Wd"], p[f"gat{l}_mg_ad"].reshape(_H, 1), False)
        w_e2, den2 = _gat_w_call(al_m[:, 0], al_gd[:, 0], mx_m[0], mx_gd[0],
                                 s_mg, d_mg, _NPG)
        num2 = _seg_rows_call(hs_m, s_mg, d_mg, w_e2, bst_mg, _NPG)
        ng = _finalize_call(num2, den2, p[f"gat{l}_mg_b"].reshape(1, _H),
                            _NTG, True)
        g, m = ng, nm

    for l in range(2):
        h_g = _h_call(g, p[f"gcn{l}_gg_W"])
        num_g = _seg_rows_call(h_g, s_gg, d_gg, w_gg, bst_gg, _NPG)
        ng = _finalize_call(num_g, None, p[f"gcn{l}_gg_b"].reshape(1, _H),
                            _NTG, False)
        h_m = _h_call(m, p[f"gcn{l}_mm_W"])
        num_m = _seg_rows_call(h_m, s_mm, d_mm, w_mm, bst_mm, _NPM)
        nm = _finalize_call(num_m, None, p[f"gcn{l}_mm_b"].reshape(1, _H),
                            _NTM, False)
        g, m = ng, nm

    return (g[:_N_GENE], m[:_N_MESH])


# speedup vs baseline: 4.4219x; 4.4219x over previous
import functools

import jax
import jax.numpy as jnp
from jax import lax
from jax.experimental import pallas as pl
from jax.experimental.pallas import tpu as pltpu
from jax.experimental.pallas import tpu_sc as plsc

_N_GENE = 20000
_N_MESH = 10000
_H = 128
_NTG = 20096
_NTM = 10112
_NPG = 20224
_NPM = 10112
_EC_QUANT = 32 * 256

_MESH = plsc.VectorSubcoreMesh(core_axis_name="c", subcore_axis_name="s")
_F32 = jnp.float32
_I32 = jnp.int32


def _wid():
    return lax.axis_index("s") * 2 + lax.axis_index("c")


def _mm_stats_call(x, w, nt):
    n, kp = x.shape
    nb = nt // 128
    tail = n - (nb - 1) * 128

    def body(x_ref, w_ref, h_ref, st_ref, acc):
        i = pl.program_id(0)

        @pl.when(i == 0)
        def _():
            acc[...] = jnp.zeros_like(acc)

        h = jnp.dot(x_ref[...], w_ref[...], preferred_element_type=_F32)

        @pl.when(i == nb - 1)
        def _():
            rows = lax.broadcasted_iota(_I32, (128, _H), 0)
            h_ref[...] = jnp.where(rows < tail, h, 0.0)
            acc[0:1, :] += jnp.sum(h_ref[...], axis=0, keepdims=True)
            acc[1:2, :] += jnp.sum(h_ref[...] * h_ref[...], axis=0,
                                   keepdims=True)
            st_ref[...] = acc[...]

        @pl.when(i < nb - 1)
        def _():
            h_ref[...] = h
            acc[0:1, :] += jnp.sum(h, axis=0, keepdims=True)
            acc[1:2, :] += jnp.sum(h * h, axis=0, keepdims=True)

    return pl.pallas_call(
        body,
        grid=(nb,),
        in_specs=[
            pl.BlockSpec((128, kp), lambda i: (i, 0)),
            pl.BlockSpec((kp, _H), lambda i: (0, 0)),
        ],
        out_specs=[
            pl.BlockSpec((128, _H), lambda i: (i, 0)),
            pl.BlockSpec((8, _H), lambda i: (0, 0)),
        ],
        out_shape=[
            jax.ShapeDtypeStruct((nt, _H), _F32),
            jax.ShapeDtypeStruct((8, _H), _F32),
        ],
        scratch_shapes=[pltpu.VMEM((8, _H), _F32)],
    )(x, w)


def _bn_apply_call(h, stats, gamma, beta, n_true):
    nt = h.shape[0]
    nb = nt // 128
    inv_n = 1.0 / float(n_true)

    def body(h_ref, st_ref, g_ref, b_ref, y_ref):
        mu = st_ref[0:1, :] * inv_n
        var = st_ref[1:2, :] * inv_n - mu * mu
        scale = g_ref[...] * lax.rsqrt(var + 1e-5)
        y = (h_ref[...] - mu) * scale + b_ref[...]
        y_ref[...] = jnp.maximum(y, 0.0)

    return pl.pallas_call(
        body,
        grid=(nb,),
        in_specs=[
            pl.BlockSpec((128, _H), lambda i: (i, 0)),
            pl.BlockSpec((8, _H), lambda i: (0, 0)),
            pl.BlockSpec((1, _H), lambda i: (0, 0)),
            pl.BlockSpec((1, _H), lambda i: (0, 0)),
        ],
        out_specs=pl.BlockSpec((128, _H), lambda i: (i, 0)),
        out_shape=jax.ShapeDtypeStruct((nt, _H), _F32),
    )(h, stats, gamma, beta)


def _h_alpha_call(y, w, a_col, want_h):
    nt = y.shape[0]
    nb = nt // 128

    def body(*refs):
        if want_h:
            y_ref, w_ref, a_ref, h_ref, al_ref, m_ref, wv, mx = refs
        else:
            y_ref, w_ref, a_ref, al_ref, m_ref, wv, mx = refs
        i = pl.program_id(0)

        @pl.when(i == 0)
        def _():
            wv[...] = jnp.dot(w_ref[...], a_ref[...],
                              preferred_element_type=_F32)
            mx[0, 0] = -3.0e38

        yb = y_ref[...]
        if want_h:
            h_ref[...] = jnp.dot(yb, w_ref[...], preferred_element_type=_F32)
        av = jnp.dot(yb, wv[...], preferred_element_type=_F32)
        al_ref[...] = av
        mx[0, 0] = jnp.maximum(mx[0, 0], jnp.max(av))
        m_ref[...] = jnp.full((1, _H), mx[0, 0], _F32)

    out_specs = [
        pl.BlockSpec((128, 1), lambda i: (i, 0)),
        pl.BlockSpec((1, _H), lambda i: (0, 0)),
    ]
    out_shape = [
        jax.ShapeDtypeStruct((nt, 1), _F32),
        jax.ShapeDtypeStruct((1, _H), _F32),
    ]
    if want_h:
        out_specs.insert(0, pl.BlockSpec((128, _H), lambda i: (i, 0)))
        out_shape.insert(0, jax.ShapeDtypeStruct((nt, _H), _F32))

    return pl.pallas_call(
        body,
        grid=(nb,),
        in_specs=[
            pl.BlockSpec((128, _H), lambda i: (i, 0)),
            pl.BlockSpec((_H, _H), lambda i: (0, 0)),
            pl.BlockSpec((_H, 1), lambda i: (0, 0)),
        ],
        out_specs=out_specs,
        out_shape=out_shape,
        scratch_shapes=[pltpu.VMEM((_H, 1), _F32), pltpu.SMEM((1, 1), _F32)],
    )(y, w, a_col)


def _h_call(y, w):
    nt = y.shape[0]
    nb = nt // 128

    def body(y_ref, w_ref, h_ref):
        h_ref[...] = jnp.dot(y_ref[...], w_ref[...],
                             preferred_element_type=_F32)

    return pl.pallas_call(
        body,
        grid=(nb,),
        in_specs=[
            pl.BlockSpec((128, _H), lambda i: (i, 0)),
            pl.BlockSpec((_H, _H), lambda i: (0, 0)),
        ],
        out_specs=pl.BlockSpec((128, _H), lambda i: (i, 0)),
        out_shape=jax.ShapeDtypeStruct((nt, _H), _F32),
    )(y, w)


def _finalize_call(num, den, b, nt, gat):
    _, p_cnt, npad, wd = num.shape
    nb = nt // 128
    ones32 = None

    in_specs = []
    for sc in range(2):
        for p in range(p_cnt):
            in_specs.append(pl.BlockSpec(
                (1, 1, 128, wd),
                functools.partial(lambda i, _sc=sc, _p=p: (_sc, _p, i, 0))))
    args = [num] * (2 * p_cnt)
    if gat:
        in_specs.append(pl.BlockSpec((32, 128), lambda i: (0, i)))
        args.append(den)
    in_specs.append(pl.BlockSpec((1, _H), lambda i: (0, 0)))
    args.append(b)

    def body(*refs):
        refs = list(refs)
        y_ref = refs.pop()
        b_ref = refs.pop()
        if gat:
            den_ref = refs.pop()
        parts = [jnp.reshape(r[...], (128, wd)) for r in refs]
        if p_cnt == 2:
            big = jnp.concatenate([parts[0] + parts[2], parts[1] + parts[3]],
                                  axis=1)
        else:
            big = parts[0] + parts[1]
        if gat:
            dcol = lax.dot_general(den_ref[...], jnp.ones((32, 1), _F32),
                                   (((0,), (0,)), ((), ())),
                                   preferred_element_type=_F32)
            big = big / (dcol + 1e-16)
        y_ref[...] = jnp.maximum(big + b_ref[...], 0.0)

    return pl.pallas_call(
        body,
        grid=(nb,),
        in_specs=in_specs,
        out_specs=pl.BlockSpec((128, _H), lambda i: (i, 0)),
        out_shape=jax.ShapeDtypeStruct((nt, _H), _F32),
    )(*args)


def _dinv_call(deg_parts):
    npad = deg_parts.shape[1]
    nb = npad // 128

    def body(d_ref, o_ref):
        deg = lax.dot_general(d_ref[...], jnp.ones((32, 1), _F32),
                              (((0,), (0,)), ((), ())),
                              preferred_element_type=_F32)
        o_ref[...] = jnp.where(deg > 0.0, lax.rsqrt(jnp.maximum(deg, 1e-12)),
                               0.0)

    return pl.pallas_call(
        body,
        grid=(nb,),
        in_specs=[pl.BlockSpec((32, 128), lambda i: (0, i))],
        out_specs=pl.BlockSpec((128, 1), lambda i: (i, 0)),
        out_shape=jax.ShapeDtypeStruct((npad, 1), _F32),
    )(deg_parts)


def _zero_1d(ref, nwords):
    z = jnp.zeros((16,), _F32)

    def bd(i, c):
        ref[pl.ds(i * 16, 16)] = z
        return c
    lax.fori_loop(0, nwords // 16, bd, 0)


def _deg_call(dst, npad):
    epad = dst.shape[0]
    ec = epad // 32

    @functools.partial(
        pl.kernel, mesh=_MESH,
        compiler_params=pltpu.CompilerParams(needs_layout_passes=False, use_tc_tiling_on_sc=False),
        out_type=jax.ShapeDtypeStruct((32, npad), _F32),
        scratch_types=[pltpu.VMEM((npad,), _F32), pltpu.VMEM((128,), _I32)],
    )
    def k(dst_hbm, out_hbm, deg_v, idx_v):
        wid = _wid()
        base = wid * ec
        _zero_1d(deg_v, npad)
        ones16 = jnp.ones((16,), _F32)

        def chunk(c, carry):
            pltpu.sync_copy(dst_hbm.at[pl.ds(base + c * 128, 128)], idx_v)
            for j in range(8):
                d16 = idx_v[pl.ds(j * 16, 16)]
                plsc.addupdate_scatter(deg_v, [d16], ones16)
            return carry
        lax.fori_loop(0, ec // 128, chunk, 0)
        pltpu.sync_copy(deg_v, out_hbm.at[wid])

    return k(dst)


def _gat_w_call(asrc, adst, msrc, mdst, src, dst, npad):
    epad = src.shape[0]
    ec = epad // 32
    ns = asrc.shape[0]
    nd = adst.shape[0]

    @functools.partial(
        pl.kernel, mesh=_MESH,
        compiler_params=pltpu.CompilerParams(needs_layout_passes=False, use_tc_tiling_on_sc=False),
        out_type=[jax.ShapeDtypeStruct((epad,), _F32),
                  jax.ShapeDtypeStruct((32, npad), _F32)],
        scratch_types=[
            pltpu.VMEM((ns,), _F32), pltpu.VMEM((nd,), _F32),
            pltpu.VMEM((npad,), _F32),
            pltpu.VMEM((128,), _I32), pltpu.VMEM((128,), _I32),
            pltpu.VMEM((128,), _F32),
            pltpu.VMEM((128,), _F32), pltpu.VMEM((128,), _F32),
        ],
    )
    def k(asrc_hbm, adst_hbm, msrc_hbm, mdst_hbm, src_hbm, dst_hbm,
          w_hbm, den_hbm, as_v, ad_v, den_v, si_v, di_v, w_v, ms_v, md_v):
        wid = _wid()
        base = wid * ec
        pltpu.sync_copy(asrc_hbm, as_v)
        pltpu.sync_copy(adst_hbm, ad_v)
        pltpu.sync_copy(msrc_hbm, ms_v)
        pltpu.sync_copy(mdst_hbm, md_v)
        _zero_1d(den_v, npad)
        msum = ms_v[pl.ds(0, 16)] + md_v[pl.ds(0, 16)]
        mb = jnp.where(msum > 0.0, msum, 0.2 * msum)

        def chunk(c, carry):
            off = base + c * 128
            pltpu.sync_copy(src_hbm.at[pl.ds(off, 128)], si_v)
            pltpu.sync_copy(dst_hbm.at[pl.ds(off, 128)], di_v)
            for j in range(8):
                s16 = si_v[pl.ds(j * 16, 16)]
                d16 = di_v[pl.ds(j * 16, 16)]
                e = plsc.load_gather(as_v, [s16]) + plsc.load_gather(ad_v, [d16])
                e = jnp.where(e > 0.0, e, 0.2 * e)
                ex = jnp.exp(e - mb)
                w_v[pl.ds(j * 16, 16)] = ex
                plsc.addupdate_scatter(den_v, [d16], ex)
            pltpu.sync_copy(w_v, w_hbm.at[pl.ds(off, 128)])
            return carry
        lax.fori_loop(0, ec // 128, chunk, 0)
        pltpu.sync_copy(den_v, den_hbm.at[wid])

    return k(asrc, adst, msrc, mdst, src, dst)


def _gcn_w_call(dinv, src, dst):
    epad = src.shape[0]
    ec = epad // 32
    nn = dinv.shape[0]

    @functools.partial(
        pl.kernel, mesh=_MESH,
        compiler_params=pltpu.CompilerParams(needs_layout_passes=False, use_tc_tiling_on_sc=False),
        out_type=jax.ShapeDtypeStruct((epad,), _F32),
        scratch_types=[
            pltpu.VMEM((nn,), _F32),
            pltpu.VMEM((128,), _I32), pltpu.VMEM((128,), _I32),
            pltpu.VMEM((128,), _F32),
        ],
    )
    def k(dinv_hbm, src_hbm, dst_hbm, w_hbm, dv, si_v, di_v, w_v):
        wid = _wid()
        base = wid * ec
        pltpu.sync_copy(dinv_hbm, dv)

        def chunk(c, carry):
            off = base + c * 128
            pltpu.sync_copy(src_hbm.at[pl.ds(off, 128)], si_v)
            pltpu.sync_copy(dst_hbm.at[pl.ds(off, 128)], di_v)
            for j in range(8):
                s16 = si_v[pl.ds(j * 16, 16)]
                d16 = di_v[pl.ds(j * 16, 16)]
                w_v[pl.ds(j * 16, 16)] = (plsc.load_gather(dv, [s16]) *
                                          plsc.load_gather(dv, [d16]))
            pltpu.sync_copy(w_v, w_hbm.at[pl.ds(off, 128)])
            return carry
        lax.fori_loop(0, ec // 128, chunk, 0)

    return k(dinv, src, dst)


def _seg_rows_call(feat, src, dst, w, npad, passes):
    epad = src.shape[0]
    ec = epad // 32
    nc = ec // 128
    wd = feat.shape[1]
    nblk = npad // 128
    nvpr = wd // 16

    @functools.partial(
        pl.kernel, mesh=_MESH,
        compiler_params=pltpu.CompilerParams(needs_layout_passes=False,
                                             use_tc_tiling_on_sc=False),
        out_type=jax.ShapeDtypeStruct((2, passes, npad, wd), _F32),
        scratch_types=[
            pltpu.VMEM_SHARED((npad, wd), _F32),
            pltpu.VMEM((128,), _I32), pltpu.VMEM((128,), _I32),
            pltpu.VMEM((128,), _I32), pltpu.VMEM((128,), _F32),
            pltpu.VMEM((128, wd), _F32), pltpu.VMEM((128, wd), _F32),
            pltpu.SemaphoreType.DMA,
        ],
    )
    def k(feat_hbm, src_hbm, dst_hbm, w_hbm, out_hbm,
          acc, si_v, di_v, gi_v, w_v, rows_v, zrow_v, sem):
        cid = lax.axis_index("c")
        sid = lax.axis_index("s")
        wid = sid * 2 + cid
        base = wid * ec
        z16 = jnp.zeros((16,), _F32)

        def zr(i, c):
            for j in range(nvpr):
                zrow_v[i, pl.ds(j * 16, 16)] = z16
            return c
        lax.fori_loop(0, 128, zr, 0)

        for p in range(passes):
            nbt = (nblk - sid + 15) // 16

            def zb(i, c):
                pltpu.sync_copy(zrow_v, acc.at[pl.ds((sid + i * 16) * 128,
                                                     128)])
                return c
            lax.fori_loop(0, nbt, zb, 0)
            plsc.subcore_barrier()

            def chunk(c, carry):
                off = base + c * 128
                pltpu.sync_copy(src_hbm.at[pl.ds(off, 128)], si_v)
                pltpu.sync_copy(dst_hbm.at[pl.ds(off, 128)], di_v)
                pltpu.sync_copy(w_hbm.at[pl.ds(off, 128)], w_v)
                if passes == 1:
                    pltpu.async_copy(feat_hbm.at[si_v], rows_v, sem).wait()
                else:
                    for j in range(8):
                        s16 = si_v[pl.ds(j * 16, 16)]
                        gi_v[pl.ds(j * 16, 16)] = s16 * passes + p
                    pltpu.async_copy(feat_hbm.at[gi_v], rows_v, sem).wait()

                def scale(i, cc):
                    for rr in range(4):
                        r = i * 4 + rr
                        wv = plsc.load_gather(w_v,
                                              [jnp.full((16,), r, _I32)])
                        for j in range(nvpr):
                            rows_v[r, pl.ds(j * 16, 16)] = (
                                rows_v[r, pl.ds(j * 16, 16)] * wv)
                    return cc
                lax.fori_loop(0, 32, scale, 0)
                pltpu.sync_copy(rows_v, acc.at[di_v], add=True)
                return carry
            lax.fori_loop(0, nc, chunk, 0)
            plsc.subcore_barrier()

            def ob_(i, c):
                o = (sid + i * 16) * 128
                pltpu.sync_copy(acc.at[pl.ds(o, 128)],
                                out_hbm.at[cid, p, pl.ds(o, 128)])
                return c
            lax.fori_loop(0, nbt, ob_, 0)
            plsc.subcore_barrier()

    return k(feat, src, dst, w)


def _pad_rows(x, nt):
    return jnp.pad(x, ((0, nt - x.shape[0]), (0, 0)))


def _pad_edges(ei, dump):
    e = ei.shape[1]
    epad = -(-e // _EC_QUANT) * _EC_QUANT
    src = jnp.pad(ei[0], (0, epad - e))
    dst = jnp.pad(ei[1], (0, epad - e), constant_values=dump)
    return src, dst


def kernel(x_gene, x_mesh, params, ei_gg, ei_mm, ei_gm, ei_mg,
           edge_label_index):
    p = params
    del edge_label_index

    hg, stg = _mm_stats_call(x_gene, p["lin_g_W"], _NTG)
    g = _bn_apply_call(hg, stg, p["bn_g_g"].reshape(1, _H),
                       p["bn_g_b"].reshape(1, _H), _N_GENE)
    hm, stm = _mm_stats_call(x_mesh, p["lin_m_W"], _NTM)
    m = _bn_apply_call(hm, stm, p["bn_m_g"].reshape(1, _H),
                       p["bn_m_b"].reshape(1, _H), _N_MESH)

    s_gg, d_gg = _pad_edges(ei_gg, _NTG - 1)
    s_mm, d_mm = _pad_edges(ei_mm, _NTM - 1)
    s_gm, d_gm = _pad_edges(ei_gm, _NTM - 1)
    s_mg, d_mg = _pad_edges(ei_mg, _NTG - 1)

    deg_gg = _deg_call(d_gg, _NPG)
    deg_mm = _deg_call(d_mm, _NPM)
    dinv_gg = _dinv_call(deg_gg)[: _NTG, 0]
    dinv_mm = _dinv_call(deg_mm)[: _NTM, 0]
    w_gg = _gcn_w_call(dinv_gg, s_gg, d_gg)
    w_mm = _gcn_w_call(dinv_mm, s_mm, d_mm)

    for l in range(2):
        hs_g, al_g, mx_g = _h_alpha_call(
            g, p[f"gat{l}_gm_Ws"], p[f"gat{l}_gm_as"].reshape(_H, 1), True)
        al_md, mx_md = _h_alpha_call(
            m, p[f"gat{l}_gm_Wd"], p[f"gat{l}_gm_ad"].reshape(_H, 1), False)
        w_e, den = _gat_w_call(al_g[:, 0], al_md[:, 0], mx_g[0], mx_md[0],
                               s_gm, d_gm, _NPM)
        num = _seg_rows_call(hs_g, s_gm, d_gm, w_e, _NPM, 1)
        nm = _finalize_call(num, den, p[f"gat{l}_gm_b"].reshape(1, _H),
                            _NTM, True)

        hs_m, al_m, mx_m = _h_alpha_call(
            m, p[f"gat{l}_mg_Ws"], p[f"gat{l}_mg_as"].reshape(_H, 1), True)
        al_gd, mx_gd = _h_alpha_call(
            g, p[f"gat{l}_mg_Wd"], p[f"gat{l}_mg_ad"].reshape(_H, 1), False)
        w_e2, den2 = _gat_w_call(al_m[:, 0], al_gd[:, 0], mx_m[0], mx_gd[0],
                                 s_mg, d_mg, _NPG)
        num2 = _seg_rows_call(hs_m.reshape(_NTM * 2, 64), s_mg, d_mg, w_e2,
                              _NPG, 2)
        ng = _finalize_call(num2, den2, p[f"gat{l}_mg_b"].reshape(1, _H),
                            _NTG, True)
        g, m = ng, nm

    for l in range(2):
        h_g = _h_call(g, p[f"gcn{l}_gg_W"])
        num_g = _seg_rows_call(h_g.reshape(_NTG * 2, 64), s_gg, d_gg, w_gg,
                               _NPG, 2)
        ng = _finalize_call(num_g, None, p[f"gcn{l}_gg_b"].reshape(1, _H),
                            _NTG, False)
        h_m = _h_call(m, p[f"gcn{l}_mm_W"])
        num_m = _seg_rows_call(h_m, s_mm, d_mm, w_mm, _NPM, 1)
        nm = _finalize_call(num_m, None, p[f"gcn{l}_mm_b"].reshape(1, _H),
                            _NTM, False)
        g, m = ng, nm

    return (g[:_N_GENE], m[:_N_MESH])


# speedup vs baseline: 5.1480x; 1.1642x over previous
import functools

import jax
import jax.numpy as jnp
from jax import lax
from jax.experimental import pallas as pl
from jax.experimental.pallas import tpu as pltpu
from jax.experimental.pallas import tpu_sc as plsc

_N_GENE = 20000
_N_MESH = 10000
_H = 128
_NTG = 20096
_NTM = 10112
_NPG = 20480
_NPM = 10240
_EC_QUANT = 32 * 128

_MESH = plsc.VectorSubcoreMesh(core_axis_name="c", subcore_axis_name="s")
_F32 = jnp.float32
_I32 = jnp.int32


def _wid():
    return lax.axis_index("s") * 2 + lax.axis_index("c")


def _mm_stats_call(x, w, n_true):
    nt, kp = x.shape
    nb = nt // 128

    def body(x_ref, w_ref, h_ref, st_ref, acc):
        i = pl.program_id(0)

        @pl.when(i == 0)
        def _():
            acc[...] = jnp.zeros_like(acc)

        h = jnp.dot(x_ref[...], w_ref[...], preferred_element_type=_F32)
        h_ref[...] = h
        acc[0:1, :] += jnp.sum(h, axis=0, keepdims=True)
        acc[1:2, :] += jnp.sum(h * h, axis=0, keepdims=True)

        @pl.when(i == nb - 1)
        def _():
            st_ref[...] = acc[...]

    return pl.pallas_call(
        body,
        grid=(nb,),
        in_specs=[
            pl.BlockSpec((128, kp), lambda i: (i, 0)),
            pl.BlockSpec((kp, _H), lambda i: (0, 0)),
        ],
        out_specs=[
            pl.BlockSpec((128, _H), lambda i: (i, 0)),
            pl.BlockSpec((8, _H), lambda i: (0, 0)),
        ],
        out_shape=[
            jax.ShapeDtypeStruct((nt, _H), _F32),
            jax.ShapeDtypeStruct((8, _H), _F32),
        ],
        scratch_shapes=[pltpu.VMEM((8, _H), _F32)],
    )(x, w)


def _bn_apply_call(h, stats, gamma, beta, n_true):
    nt = h.shape[0]
    nb = nt // 128
    inv_n = 1.0 / float(n_true)

    def body(h_ref, st_ref, g_ref, b_ref, y_ref):
        mu = st_ref[0:1, :] * inv_n
        var = st_ref[1:2, :] * inv_n - mu * mu
        scale = g_ref[...] * lax.rsqrt(var + 1e-5)
        y = (h_ref[...] - mu) * scale + b_ref[...]
        y_ref[...] = jnp.maximum(y, 0.0)

    return pl.pallas_call(
        body,
        grid=(nb,),
        in_specs=[
            pl.BlockSpec((128, _H), lambda i: (i, 0)),
            pl.BlockSpec((8, _H), lambda i: (0, 0)),
            pl.BlockSpec((1, _H), lambda i: (0, 0)),
            pl.BlockSpec((1, _H), lambda i: (0, 0)),
        ],
        out_specs=pl.BlockSpec((128, _H), lambda i: (i, 0)),
        out_shape=jax.ShapeDtypeStruct((nt, _H), _F32),
    )(h, stats, gamma, beta)


def _h_alpha_call(y, w, a_col, want_h):
    nt = y.shape[0]
    nb = nt // 128

    def body(*refs):
        if want_h:
            y_ref, w_ref, a_ref, h_ref, al_ref, m_ref, wv, mx = refs
        else:
            y_ref, w_ref, a_ref, al_ref, m_ref, wv, mx = refs
        i = pl.program_id(0)

        @pl.when(i == 0)
        def _():
            wv[...] = jnp.dot(w_ref[...], a_ref[...],
                              preferred_element_type=_F32)
            mx[0, 0] = -3.0e38

        yb = y_ref[...]
        if want_h:
            h_ref[...] = jnp.dot(yb, w_ref[...], preferred_element_type=_F32)
        av = jnp.dot(yb, wv[...], preferred_element_type=_F32)
        al_ref[...] = av
        mx[0, 0] = jnp.maximum(mx[0, 0], jnp.max(av))
        m_ref[...] = jnp.full((1, _H), mx[0, 0], _F32)

    out_specs = [
        pl.BlockSpec((128, 1), lambda i: (i, 0)),
        pl.BlockSpec((1, _H), lambda i: (0, 0)),
    ]
    out_shape = [
        jax.ShapeDtypeStruct((nt, 1), _F32),
        jax.ShapeDtypeStruct((1, _H), _F32),
    ]
    if want_h:
        out_specs.insert(0, pl.BlockSpec((128, _H), lambda i: (i, 0)))
        out_shape.insert(0, jax.ShapeDtypeStruct((nt, _H), _F32))

    return pl.pallas_call(
        body,
        grid=(nb,),
        in_specs=[
            pl.BlockSpec((128, _H), lambda i: (i, 0)),
            pl.BlockSpec((_H, _H), lambda i: (0, 0)),
            pl.BlockSpec((_H, 1), lambda i: (0, 0)),
        ],
        out_specs=out_specs,
        out_shape=out_shape,
        scratch_shapes=[pltpu.VMEM((_H, 1), _F32), pltpu.SMEM((1, 1), _F32)],
    )(y, w, a_col)


def _h_call(y, w):
    nt = y.shape[0]
    nb = nt // 128

    def body(y_ref, w_ref, h_ref):
        h_ref[...] = jnp.dot(y_ref[...], w_ref[...],
                             preferred_element_type=_F32)

    return pl.pallas_call(
        body,
        grid=(nb,),
        in_specs=[
            pl.BlockSpec((128, _H), lambda i: (i, 0)),
            pl.BlockSpec((_H, _H), lambda i: (0, 0)),
        ],
        out_specs=pl.BlockSpec((128, _H), lambda i: (i, 0)),
        out_shape=jax.ShapeDtypeStruct((nt, _H), _F32),
    )(y, w)


def _finalize_call(num, den, b, nt, gat):
    _, p_cnt, npad, wd = num.shape
    nb = nt // 128
    ones32 = None

    in_specs = []
    for sc in range(2):
        for p in range(p_cnt):
            in_specs.append(pl.BlockSpec(
                (1, 1, 128, wd),
                functools.partial(lambda i, _sc=sc, _p=p: (_sc, _p, i, 0))))
    args = [num] * (2 * p_cnt)
    if gat:
        in_specs.append(pl.BlockSpec((32, 128), lambda i: (0, i)))
        args.append(den)
    in_specs.append(pl.BlockSpec((1, _H), lambda i: (0, 0)))
    args.append(b)

    def body(*refs):
        refs = list(refs)
        y_ref = refs.pop()
        b_ref = refs.pop()
        if gat:
            den_ref = refs.pop()
        parts = [jnp.reshape(r[...], (128, wd)) for r in refs]
        if p_cnt == 2:
            big = jnp.concatenate([parts[0] + parts[2], parts[1] + parts[3]],
                                  axis=1)
        else:
            big = parts[0] + parts[1]
        if gat:
            dcol = lax.dot_general(den_ref[...], jnp.ones((32, 1), _F32),
                                   (((0,), (0,)), ((), ())),
                                   preferred_element_type=_F32)
            big = big / (dcol + 1e-16)
        y_ref[...] = jnp.maximum(big + b_ref[...], 0.0)

    return pl.pallas_call(
        body,
        grid=(nb,),
        in_specs=in_specs,
        out_specs=pl.BlockSpec((128, _H), lambda i: (i, 0)),
        out_shape=jax.ShapeDtypeStruct((nt, _H), _F32),
    )(*args)


def _dinv_call(deg_parts):
    npad = deg_parts.shape[1]
    nb = npad // 128

    def body(d_ref, o_ref):
        deg = lax.dot_general(d_ref[...], jnp.ones((32, 1), _F32),
                              (((0,), (0,)), ((), ())),
                              preferred_element_type=_F32)
        o_ref[...] = jnp.where(deg > 0.0, lax.rsqrt(jnp.maximum(deg, 1e-12)),
                               0.0)

    return pl.pallas_call(
        body,
        grid=(nb,),
        in_specs=[pl.BlockSpec((32, 128), lambda i: (0, i))],
        out_specs=pl.BlockSpec((128, 1), lambda i: (i, 0)),
        out_shape=jax.ShapeDtypeStruct((npad, 1), _F32),
    )(deg_parts)


def _zero_1d(ref, nwords):
    z = jnp.zeros((16,), _F32)

    def bd(i, c):
        ref[pl.ds(i * 16, 16)] = z
        return c
    lax.fori_loop(0, nwords // 16, bd, 0)


def _deg_call(dst, npad):
    epad = dst.shape[0]
    ec = epad // 32

    @functools.partial(
        pl.kernel, mesh=_MESH,
        compiler_params=pltpu.CompilerParams(needs_layout_passes=False, use_tc_tiling_on_sc=False),
        out_type=jax.ShapeDtypeStruct((32, npad), _F32),
        scratch_types=[pltpu.VMEM((npad,), _F32), pltpu.VMEM((128,), _I32)],
    )
    def k(dst_hbm, out_hbm, deg_v, idx_v):
        wid = _wid()
        base = wid * ec
        _zero_1d(deg_v, npad)
        ones16 = jnp.ones((16,), _F32)

        def chunk(c, carry):
            pltpu.sync_copy(dst_hbm.at[pl.ds(base + c * 128, 128)], idx_v)
            for j in range(8):
                d16 = idx_v[pl.ds(j * 16, 16)]
                plsc.addupdate_scatter(deg_v, [d16], ones16)
            return carry
        lax.fori_loop(0, ec // 128, chunk, 0)
        pltpu.sync_copy(deg_v, out_hbm.at[wid])

    return k(dst)


def _gat_w_call(asrc, adst, msrc, mdst, src, dst, npad):
    epad = src.shape[0]
    ec = epad // 32
    ns = asrc.shape[0]
    nd = adst.shape[0]

    @functools.partial(
        pl.kernel, mesh=_MESH,
        compiler_params=pltpu.CompilerParams(needs_layout_passes=False, use_tc_tiling_on_sc=False),
        out_type=[jax.ShapeDtypeStruct((epad,), _F32),
                  jax.ShapeDtypeStruct((32, npad), _F32)],
        scratch_types=[
            pltpu.VMEM((ns,), _F32), pltpu.VMEM((nd,), _F32),
            pltpu.VMEM((npad,), _F32),
            pltpu.VMEM((128,), _I32), pltpu.VMEM((128,), _I32),
            pltpu.VMEM((128,), _F32),
            pltpu.VMEM((128,), _F32), pltpu.VMEM((128,), _F32),
        ],
    )
    def k(asrc_hbm, adst_hbm, msrc_hbm, mdst_hbm, src_hbm, dst_hbm,
          w_hbm, den_hbm, as_v, ad_v, den_v, si_v, di_v, w_v, ms_v, md_v):
        wid = _wid()
        base = wid * ec
        pltpu.sync_copy(asrc_hbm, as_v)
        pltpu.sync_copy(adst_hbm, ad_v)
        pltpu.sync_copy(msrc_hbm, ms_v)
        pltpu.sync_copy(mdst_hbm, md_v)
        _zero_1d(den_v, npad)
        msum = ms_v[pl.ds(0, 16)] + md_v[pl.ds(0, 16)]
        mb = jnp.where(msum > 0.0, msum, 0.2 * msum)

        def chunk(c, carry):
            off = base + c * 128
            pltpu.sync_copy(src_hbm.at[pl.ds(off, 128)], si_v)
            pltpu.sync_copy(dst_hbm.at[pl.ds(off, 128)], di_v)
            for j in range(8):
                s16 = si_v[pl.ds(j * 16, 16)]
                d16 = di_v[pl.ds(j * 16, 16)]
                e = plsc.load_gather(as_v, [s16]) + plsc.load_gather(ad_v, [d16])
                e = jnp.where(e > 0.0, e, 0.2 * e)
                ex = jnp.exp(e - mb)
                w_v[pl.ds(j * 16, 16)] = ex
                plsc.addupdate_scatter(den_v, [d16], ex)
            pltpu.sync_copy(w_v, w_hbm.at[pl.ds(off, 128)])
            return carry
        lax.fori_loop(0, ec // 128, chunk, 0)
        pltpu.sync_copy(den_v, den_hbm.at[wid])

    return k(asrc, adst, msrc, mdst, src, dst)


def _gcn_w_call(dinv, src, dst):
    epad = src.shape[0]
    ec = epad // 32
    nn = dinv.shape[0]

    @functools.partial(
        pl.kernel, mesh=_MESH,
        compiler_params=pltpu.CompilerParams(needs_layout_passes=False, use_tc_tiling_on_sc=False),
        out_type=jax.ShapeDtypeStruct((epad,), _F32),
        scratch_types=[
            pltpu.VMEM((nn,), _F32),
            pltpu.VMEM((128,), _I32), pltpu.VMEM((128,), _I32),
            pltpu.VMEM((128,), _F32),
        ],
    )
    def k(dinv_hbm, src_hbm, dst_hbm, w_hbm, dv, si_v, di_v, w_v):
        wid = _wid()
        base = wid * ec
        pltpu.sync_copy(dinv_hbm, dv)

        def chunk(c, carry):
            off = base + c * 128
            pltpu.sync_copy(src_hbm.at[pl.ds(off, 128)], si_v)
            pltpu.sync_copy(dst_hbm.at[pl.ds(off, 128)], di_v)
            for j in range(8):
                s16 = si_v[pl.ds(j * 16, 16)]
                d16 = di_v[pl.ds(j * 16, 16)]
                w_v[pl.ds(j * 16, 16)] = (plsc.load_gather(dv, [s16]) *
                                          plsc.load_gather(dv, [d16]))
            pltpu.sync_copy(w_v, w_hbm.at[pl.ds(off, 128)])
            return carry
        lax.fori_loop(0, ec // 128, chunk, 0)

    return k(dinv, src, dst)


def _seg_rows_call(feat, src, dst, w, npad, passes):
    epad = src.shape[0]
    ec = epad // 32
    nc = ec // 128
    wd = feat.shape[1]
    nvpr = wd // 16

    @functools.partial(
        pl.kernel, mesh=_MESH,
        compiler_params=pltpu.CompilerParams(needs_layout_passes=False,
                                             use_tc_tiling_on_sc=False),
        out_type=jax.ShapeDtypeStruct((2, passes, npad, wd), _F32),
        scratch_types=[
            pltpu.VMEM_SHARED((npad, wd), _F32),
            pltpu.VMEM((128,), _I32), pltpu.VMEM((128,), _I32),
            pltpu.VMEM((128,), _I32), pltpu.VMEM((128,), _F32),
            pltpu.VMEM((128, wd), _F32), pltpu.VMEM((128, wd), _F32),
            pltpu.SemaphoreType.DMA,
        ],
    )
    def k(feat_hbm, src_hbm, dst_hbm, w_hbm, out_hbm,
          acc, si_v, di_v, gi_v, w_v, rows_v, zrow_v, sem):
        cid = lax.axis_index("c")
        sid = lax.axis_index("s")
        wid = sid * 2 + cid
        base = wid * ec
        z16 = jnp.zeros((16,), _F32)

        def zr(i, c):
            for j in range(nvpr):
                zrow_v[i, pl.ds(j * 16, 16)] = z16
            return c
        lax.fori_loop(0, 128, zr, 0)

        stripe = npad // 16
        for p in range(passes):
            for t in range(stripe // 128):
                pltpu.sync_copy(zrow_v, acc.at[pl.ds(sid * stripe + t * 128,
                                                     128)])
            plsc.subcore_barrier()

            def chunk(c, carry):
                off = base + c * 128
                pltpu.sync_copy(src_hbm.at[pl.ds(off, 128)], si_v)
                pltpu.sync_copy(dst_hbm.at[pl.ds(off, 128)], di_v)
                pltpu.sync_copy(w_hbm.at[pl.ds(off, 128)], w_v)
                if passes == 1:
                    pltpu.async_copy(feat_hbm.at[si_v], rows_v, sem).wait()
                else:
                    for j in range(8):
                        s16 = si_v[pl.ds(j * 16, 16)]
                        gi_v[pl.ds(j * 16, 16)] = s16 * passes + p
                    pltpu.async_copy(feat_hbm.at[gi_v], rows_v, sem).wait()

                def scale(i, cc):
                    for rr in range(4):
                        r = i * 4 + rr
                        wv = plsc.load_gather(w_v,
                                              [jnp.full((16,), r, _I32)])
                        for j in range(nvpr):
                            rows_v[r, pl.ds(j * 16, 16)] = (
                                rows_v[r, pl.ds(j * 16, 16)] * wv)
                    return cc
                lax.fori_loop(0, 32, scale, 0)
                pltpu.sync_copy(rows_v, acc.at[di_v], add=True)
                return carry
            lax.fori_loop(0, nc, chunk, 0)
            plsc.subcore_barrier()
            pltpu.sync_copy(acc.at[pl.ds(sid * stripe, stripe)],
                            out_hbm.at[cid, p, pl.ds(sid * stripe, stripe)])
            plsc.subcore_barrier()

    return k(feat, src, dst, w)


def _pad_rows(x, nt):
    return jnp.pad(x, ((0, nt - x.shape[0]), (0, 0)))


def _pad_edges(ei, dump):
    e = ei.shape[1]
    epad = -(-e // _EC_QUANT) * _EC_QUANT
    src = jnp.pad(ei[0], (0, epad - e))
    dst = jnp.pad(ei[1], (0, epad - e), constant_values=dump)
    return src, dst


def kernel(x_gene, x_mesh, params, ei_gg, ei_mm, ei_gm, ei_mg,
           edge_label_index):
    p = params
    del edge_label_index

    xg = jnp.pad(x_gene, ((0, _NTG - _N_GENE), (0, 20)))
    wg = jnp.pad(p["lin_g_W"], ((0, 20), (0, 0)))
    hg, stg = _mm_stats_call(xg, wg, _N_GENE)
    g = _bn_apply_call(hg, stg, p["bn_g_g"].reshape(1, _H),
                       p["bn_g_b"].reshape(1, _H), _N_GENE)
    xm = _pad_rows(x_mesh, _NTM)
    hm, stm = _mm_stats_call(xm, p["lin_m_W"], _N_MESH)
    m = _bn_apply_call(hm, stm, p["bn_m_g"].reshape(1, _H),
                       p["bn_m_b"].reshape(1, _H), _N_MESH)

    s_gg, d_gg = _pad_edges(ei_gg, _NTG - 1)
    s_mm, d_mm = _pad_edges(ei_mm, _NTM - 1)
    s_gm, d_gm = _pad_edges(ei_gm, _NTM - 1)
    s_mg, d_mg = _pad_edges(ei_mg, _NTG - 1)

    deg_gg = _deg_call(d_gg, _NPG)
    deg_mm = _deg_call(d_mm, _NPM)
    dinv_gg = _dinv_call(deg_gg)[: _NTG, 0]
    dinv_mm = _dinv_call(deg_mm)[: _NTM, 0]
    w_gg = _gcn_w_call(dinv_gg, s_gg, d_gg)
    w_mm = _gcn_w_call(dinv_mm, s_mm, d_mm)

    for l in range(2):
        hs_g, al_g, mx_g = _h_alpha_call(
            g, p[f"gat{l}_gm_Ws"], p[f"gat{l}_gm_as"].reshape(_H, 1), True)
        al_md, mx_md = _h_alpha_call(
            m, p[f"gat{l}_gm_Wd"], p[f"gat{l}_gm_ad"].reshape(_H, 1), False)
        w_e, den = _gat_w_call(al_g[:, 0], al_md[:, 0], mx_g[0], mx_md[0],
                               s_gm, d_gm, _NPM)
        num = _seg_rows_call(hs_g, s_gm, d_gm, w_e, _NPM, 1)
        nm = _finalize_call(num, den, p[f"gat{l}_gm_b"].reshape(1, _H),
                            _NTM, True)

        hs_m, al_m, mx_m = _h_alpha_call(
            m, p[f"gat{l}_mg_Ws"], p[f"gat{l}_mg_as"].reshape(_H, 1), True)
        al_gd, mx_gd = _h_alpha_call(
            g, p[f"gat{l}_mg_Wd"], p[f"gat{l}_mg_ad"].reshape(_H, 1), False)
        w_e2, den2 = _gat_w_call(al_m[:, 0], al_gd[:, 0], mx_m[0], mx_gd[0],
                                 s_mg, d_mg, _NPG)
        num2 = _seg_rows_call(hs_m.reshape(_NTM * 2, 64), s_mg, d_mg, w_e2,
                              _NPG, 2)
        ng = _finalize_call(num2, den2, p[f"gat{l}_mg_b"].reshape(1, _H),
                            _NTG, True)
        g, m = ng, nm

    for l in range(2):
        h_g = _h_call(g, p[f"gcn{l}_gg_W"])
        num_g = _seg_rows_call(h_g.reshape(_NTG * 2, 64), s_gg, d_gg, w_gg,
                               _NPG, 2)
        ng = _finalize_call(num_g, None, p[f"gcn{l}_gg_b"].reshape(1, _H),
                            _NTG, False)
        h_m = _h_call(m, p[f"gcn{l}_mm_W"])
        num_m = _seg_rows_call(h_m, s_mm, d_mm, w_mm, _NPM, 1)
        nm = _finalize_call(num_m, None, p[f"gcn{l}_mm_b"].reshape(1, _H),
                            _NTM, False)
        g, m = ng, nm

    return (g[:_N_GENE], m[:_N_MESH])


# speedup vs baseline: 5.8463x; 1.1357x over previous
import functools

import jax
import jax.numpy as jnp
from jax import lax
from jax.experimental import pallas as pl
from jax.experimental.pallas import tpu as pltpu
from jax.experimental.pallas import tpu_sc as plsc

_N_GENE = 20000
_N_MESH = 10000
_H = 128
_NTG = 20096
_NTM = 10112
_NPG = 20224
_NPM = 10112
_EC_QUANT = 32 * 128

_MESH = plsc.VectorSubcoreMesh(core_axis_name="c", subcore_axis_name="s")
_F32 = jnp.float32
_I32 = jnp.int32


def _wid():
    return lax.axis_index("s") * 2 + lax.axis_index("c")


def _mm_stats_call(x, w, n_true):
    nt, kp = x.shape
    nb = nt // 128

    def body(x_ref, w_ref, h_ref, st_ref, acc):
        i = pl.program_id(0)

        @pl.when(i == 0)
        def _():
            acc[...] = jnp.zeros_like(acc)

        h = jnp.dot(x_ref[...], w_ref[...], preferred_element_type=_F32)
        h_ref[...] = h
        acc[0:1, :] += jnp.sum(h, axis=0, keepdims=True)
        acc[1:2, :] += jnp.sum(h * h, axis=0, keepdims=True)

        @pl.when(i == nb - 1)
        def _():
            st_ref[...] = acc[...]

    return pl.pallas_call(
        body,
        grid=(nb,),
        in_specs=[
            pl.BlockSpec((128, kp), lambda i: (i, 0)),
            pl.BlockSpec((kp, _H), lambda i: (0, 0)),
        ],
        out_specs=[
            pl.BlockSpec((128, _H), lambda i: (i, 0)),
            pl.BlockSpec((8, _H), lambda i: (0, 0)),
        ],
        out_shape=[
            jax.ShapeDtypeStruct((nt, _H), _F32),
            jax.ShapeDtypeStruct((8, _H), _F32),
        ],
        scratch_shapes=[pltpu.VMEM((8, _H), _F32)],
    )(x, w)


def _bn_apply_call(h, stats, gamma, beta, n_true):
    nt = h.shape[0]
    nb = nt // 128
    inv_n = 1.0 / float(n_true)

    def body(h_ref, st_ref, g_ref, b_ref, y_ref):
        mu = st_ref[0:1, :] * inv_n
        var = st_ref[1:2, :] * inv_n - mu * mu
        scale = g_ref[...] * lax.rsqrt(var + 1e-5)
        y = (h_ref[...] - mu) * scale + b_ref[...]
        y_ref[...] = jnp.maximum(y, 0.0)

    return pl.pallas_call(
        body,
        grid=(nb,),
        in_specs=[
            pl.BlockSpec((128, _H), lambda i: (i, 0)),
            pl.BlockSpec((8, _H), lambda i: (0, 0)),
            pl.BlockSpec((1, _H), lambda i: (0, 0)),
            pl.BlockSpec((1, _H), lambda i: (0, 0)),
        ],
        out_specs=pl.BlockSpec((128, _H), lambda i: (i, 0)),
        out_shape=jax.ShapeDtypeStruct((nt, _H), _F32),
    )(h, stats, gamma, beta)


def _h_alpha_call(y, w, a_col, want_h):
    nt = y.shape[0]
    nb = nt // 128

    def body(*refs):
        if want_h:
            y_ref, w_ref, a_ref, h_ref, al_ref, m_ref, wv, mx = refs
        else:
            y_ref, w_ref, a_ref, al_ref, m_ref, wv, mx = refs
        i = pl.program_id(0)

        @pl.when(i == 0)
        def _():
            wv[...] = jnp.dot(w_ref[...], a_ref[...],
                              preferred_element_type=_F32)
            mx[0, 0] = -3.0e38

        yb = y_ref[...]
        if want_h:
            h_ref[...] = jnp.dot(yb, w_ref[...], preferred_element_type=_F32)
        av = jnp.dot(yb, wv[...], preferred_element_type=_F32)
        al_ref[...] = av
        mx[0, 0] = jnp.maximum(mx[0, 0], jnp.max(av))
        m_ref[...] = jnp.full((1, _H), mx[0, 0], _F32)

    out_specs = [
        pl.BlockSpec((128, 1), lambda i: (i, 0)),
        pl.BlockSpec((1, _H), lambda i: (0, 0)),
    ]
    out_shape = [
        jax.ShapeDtypeStruct((nt, 1), _F32),
        jax.ShapeDtypeStruct((1, _H), _F32),
    ]
    if want_h:
        out_specs.insert(0, pl.BlockSpec((128, _H), lambda i: (i, 0)))
        out_shape.insert(0, jax.ShapeDtypeStruct((nt, _H), _F32))

    return pl.pallas_call(
        body,
        grid=(nb,),
        in_specs=[
            pl.BlockSpec((128, _H), lambda i: (i, 0)),
            pl.BlockSpec((_H, _H), lambda i: (0, 0)),
            pl.BlockSpec((_H, 1), lambda i: (0, 0)),
        ],
        out_specs=out_specs,
        out_shape=out_shape,
        scratch_shapes=[pltpu.VMEM((_H, 1), _F32), pltpu.SMEM((1, 1), _F32)],
    )(y, w, a_col)


def _h_call(y, w):
    nt = y.shape[0]
    nb = nt // 128

    def body(y_ref, w_ref, h_ref):
        h_ref[...] = jnp.dot(y_ref[...], w_ref[...],
                             preferred_element_type=_F32)

    return pl.pallas_call(
        body,
        grid=(nb,),
        in_specs=[
            pl.BlockSpec((128, _H), lambda i: (i, 0)),
            pl.BlockSpec((_H, _H), lambda i: (0, 0)),
        ],
        out_specs=pl.BlockSpec((128, _H), lambda i: (i, 0)),
        out_shape=jax.ShapeDtypeStruct((nt, _H), _F32),
    )(y, w)


def _finalize_call(num, den, b, nt, gat):
    _, p_cnt, npad, wd = num.shape
    nb = nt // 128
    ones32 = None

    in_specs = []
    for sc in range(2):
        for p in range(p_cnt):
            in_specs.append(pl.BlockSpec(
                (1, 1, 128, wd),
                functools.partial(lambda i, _sc=sc, _p=p: (_sc, _p, i, 0))))
    args = [num] * (2 * p_cnt)
    if gat:
        in_specs.append(pl.BlockSpec((32, 128), lambda i: (0, i)))
        args.append(den)
    in_specs.append(pl.BlockSpec((1, _H), lambda i: (0, 0)))
    args.append(b)

    def body(*refs):
        refs = list(refs)
        y_ref = refs.pop()
        b_ref = refs.pop()
        if gat:
            den_ref = refs.pop()
        parts = [jnp.reshape(r[...], (128, wd)) for r in refs]
        if p_cnt == 2:
            big = jnp.concatenate([parts[0] + parts[2], parts[1] + parts[3]],
                                  axis=1)
        else:
            big = parts[0] + parts[1]
        if gat:
            dcol = lax.dot_general(den_ref[...], jnp.ones((32, 1), _F32),
                                   (((0,), (0,)), ((), ())),
                                   preferred_element_type=_F32)
            big = big / (dcol + 1e-16)
        y_ref[...] = jnp.maximum(big + b_ref[...], 0.0)

    return pl.pallas_call(
        body,
        grid=(nb,),
        in_specs=in_specs,
        out_specs=pl.BlockSpec((128, _H), lambda i: (i, 0)),
        out_shape=jax.ShapeDtypeStruct((nt, _H), _F32),
    )(*args)


def _dinv_call(deg_parts):
    npad = deg_parts.shape[1]
    nb = npad // 128

    def body(d_ref, o_ref):
        deg = lax.dot_general(d_ref[...], jnp.ones((32, 1), _F32),
                              (((0,), (0,)), ((), ())),
                              preferred_element_type=_F32)
        o_ref[...] = jnp.where(deg > 0.0, lax.rsqrt(jnp.maximum(deg, 1e-12)),
                               0.0)

    return pl.pallas_call(
        body,
        grid=(nb,),
        in_specs=[pl.BlockSpec((32, 128), lambda i: (0, i))],
        out_specs=pl.BlockSpec((128, 1), lambda i: (i, 0)),
        out_shape=jax.ShapeDtypeStruct((npad, 1), _F32),
    )(deg_parts)


def _zero_1d(ref, nwords):
    z = jnp.zeros((16,), _F32)

    def bd(i, c):
        ref[pl.ds(i * 16, 16)] = z
        return c
    lax.fori_loop(0, nwords // 16, bd, 0)


def _deg_call(dst, npad):
    epad = dst.shape[0]
    ec = epad // 32

    @functools.partial(
        pl.kernel, mesh=_MESH,
        compiler_params=pltpu.CompilerParams(needs_layout_passes=False, use_tc_tiling_on_sc=False),
        out_type=jax.ShapeDtypeStruct((32, npad), _F32),
        scratch_types=[pltpu.VMEM((npad,), _F32), pltpu.VMEM((128,), _I32)],
    )
    def k(dst_hbm, out_hbm, deg_v, idx_v):
        wid = _wid()
        base = wid * ec
        _zero_1d(deg_v, npad)
        ones16 = jnp.ones((16,), _F32)

        def chunk(c, carry):
            pltpu.sync_copy(dst_hbm.at[pl.ds(base + c * 128, 128)], idx_v)
            for j in range(8):
                d16 = idx_v[pl.ds(j * 16, 16)]
                plsc.addupdate_scatter(deg_v, [d16], ones16)
            return carry
        lax.fori_loop(0, ec // 128, chunk, 0)
        pltpu.sync_copy(deg_v, out_hbm.at[wid])

    return k(dst)


def _gat_w_call(asrc, adst, msrc, mdst, src, dst, npad):
    epad = src.shape[0]
    ec = epad // 32
    ns = asrc.shape[0]
    nd = adst.shape[0]

    @functools.partial(
        pl.kernel, mesh=_MESH,
        compiler_params=pltpu.CompilerParams(needs_layout_passes=False, use_tc_tiling_on_sc=False),
        out_type=[jax.ShapeDtypeStruct((epad,), _F32),
                  jax.ShapeDtypeStruct((32, npad), _F32)],
        scratch_types=[
            pltpu.VMEM((ns,), _F32), pltpu.VMEM((nd,), _F32),
            pltpu.VMEM((npad,), _F32),
            pltpu.VMEM((128,), _I32), pltpu.VMEM((128,), _I32),
            pltpu.VMEM((128,), _F32),
            pltpu.VMEM((128,), _F32), pltpu.VMEM((128,), _F32),
        ],
    )
    def k(asrc_hbm, adst_hbm, msrc_hbm, mdst_hbm, src_hbm, dst_hbm,
          w_hbm, den_hbm, as_v, ad_v, den_v, si_v, di_v, w_v, ms_v, md_v):
        wid = _wid()
        base = wid * ec
        pltpu.sync_copy(asrc_hbm, as_v)
        pltpu.sync_copy(adst_hbm, ad_v)
        pltpu.sync_copy(msrc_hbm, ms_v)
        pltpu.sync_copy(mdst_hbm, md_v)
        _zero_1d(den_v, npad)
        msum = ms_v[pl.ds(0, 16)] + md_v[pl.ds(0, 16)]
        mb = jnp.where(msum > 0.0, msum, 0.2 * msum)

        def chunk(c, carry):
            off = base + c * 128
            pltpu.sync_copy(src_hbm.at[pl.ds(off, 128)], si_v)
            pltpu.sync_copy(dst_hbm.at[pl.ds(off, 128)], di_v)
            for j in range(8):
                s16 = si_v[pl.ds(j * 16, 16)]
                d16 = di_v[pl.ds(j * 16, 16)]
                e = plsc.load_gather(as_v, [s16]) + plsc.load_gather(ad_v, [d16])
                e = jnp.where(e > 0.0, e, 0.2 * e)
                ex = jnp.exp(e - mb)
                w_v[pl.ds(j * 16, 16)] = ex
                plsc.addupdate_scatter(den_v, [d16], ex)
            pltpu.sync_copy(w_v, w_hbm.at[pl.ds(off, 128)])
            return carry
        lax.fori_loop(0, ec // 128, chunk, 0)
        pltpu.sync_copy(den_v, den_hbm.at[wid])

    return k(asrc, adst, msrc, mdst, src, dst)


def _gcn_w_call(dinv, src, dst):
    epad = src.shape[0]
    ec = epad // 32
    nn = dinv.shape[0]

    @functools.partial(
        pl.kernel, mesh=_MESH,
        compiler_params=pltpu.CompilerParams(needs_layout_passes=False, use_tc_tiling_on_sc=False),
        out_type=jax.ShapeDtypeStruct((epad,), _F32),
        scratch_types=[
            pltpu.VMEM((nn,), _F32),
            pltpu.VMEM((128,), _I32), pltpu.VMEM((128,), _I32),
            pltpu.VMEM((128,), _F32),
        ],
    )
    def k(dinv_hbm, src_hbm, dst_hbm, w_hbm, dv, si_v, di_v, w_v):
        wid = _wid()
        base = wid * ec
        pltpu.sync_copy(dinv_hbm, dv)

        def chunk(c, carry):
            off = base + c * 128
            pltpu.sync_copy(src_hbm.at[pl.ds(off, 128)], si_v)
            pltpu.sync_copy(dst_hbm.at[pl.ds(off, 128)], di_v)
            for j in range(8):
                s16 = si_v[pl.ds(j * 16, 16)]
                d16 = di_v[pl.ds(j * 16, 16)]
                w_v[pl.ds(j * 16, 16)] = (plsc.load_gather(dv, [s16]) *
                                          plsc.load_gather(dv, [d16]))
            pltpu.sync_copy(w_v, w_hbm.at[pl.ds(off, 128)])
            return carry
        lax.fori_loop(0, ec // 128, chunk, 0)

    return k(dinv, src, dst)


def _seg_rows_call(feat, src, dst, w, npad, passes):
    epad = src.shape[0]
    ec = epad // 32
    nc = ec // 128
    wd = feat.shape[1]
    nvpr = wd // 16

    @functools.partial(
        pl.kernel, mesh=_MESH,
        compiler_params=pltpu.CompilerParams(needs_layout_passes=False,
                                             use_tc_tiling_on_sc=False),
        out_type=jax.ShapeDtypeStruct((2, passes, npad, wd), _F32),
        scratch_types=[
            pltpu.VMEM_SHARED((npad, wd), _F32),
            pltpu.VMEM((2, 128), _I32), pltpu.VMEM((2, 128), _I32),
            pltpu.VMEM((128,), _I32), pltpu.VMEM((128,), _F32),
            pltpu.VMEM((2, 128, wd), _F32), pltpu.VMEM((128, wd), _F32),
            pltpu.SemaphoreType.DMA, pltpu.SemaphoreType.DMA,
        ],
    )
    def k(feat_hbm, src_hbm, dst_hbm, w_hbm, out_hbm,
          acc, si2, gi2, di_v, w_v, rows2, zrow_v, gsem0, gsem1):
        gsem = (gsem0, gsem1)
        cid = lax.axis_index("c")
        sid = lax.axis_index("s")
        wid = sid * 2 + cid
        base = wid * ec
        z16 = jnp.zeros((16,), _F32)

        def zr(i, c):
            for j in range(nvpr):
                zrow_v[i, pl.ds(j * 16, 16)] = z16
            return c
        lax.fori_loop(0, 128, zr, 0)

        nblk = npad // 128
        nbt = (nblk - sid + 15) // 16
        for p in range(passes):

            def prefetch(c, bk):
                off = base + c * 128
                pltpu.sync_copy(src_hbm.at[pl.ds(off, 128)], si2.at[bk])
                if passes == 1:
                    pltpu.async_copy(feat_hbm.at[si2.at[bk]], rows2.at[bk],
                                     gsem[bk])
                else:
                    for j in range(8):
                        s16 = si2[bk, pl.ds(j * 16, 16)]
                        gi2[bk, pl.ds(j * 16, 16)] = s16 * passes + p
                    pltpu.async_copy(feat_hbm.at[gi2.at[bk]], rows2.at[bk],
                                     gsem[bk])

            def process(c, bk):
                off = base + c * 128
                pltpu.sync_copy(dst_hbm.at[pl.ds(off, 128)], di_v)
                pltpu.sync_copy(w_hbm.at[pl.ds(off, 128)], w_v)
                pltpu.make_async_copy(feat_hbm.at[pl.ds(0, 128)],
                                      rows2.at[bk], gsem[bk]).wait()

                @pl.when(c + 1 < nc)
                def _():
                    prefetch(c + 1, 1 - bk)

                def scale(i, cc):
                    for rr in range(4):
                        r = i * 4 + rr
                        wv = plsc.load_gather(w_v,
                                              [jnp.full((16,), r, _I32)])
                        for j in range(nvpr):
                            rows2[bk, r, pl.ds(j * 16, 16)] = (
                                rows2[bk, r, pl.ds(j * 16, 16)] * wv)
                    return cc
                lax.fori_loop(0, 32, scale, 0)
                pltpu.sync_copy(rows2.at[bk], acc.at[di_v], add=True)

            def zb(i, c):
                pltpu.sync_copy(zrow_v, acc.at[pl.ds((sid + i * 16) * 128,
                                                     128)])
                return c
            lax.fori_loop(0, nbt, zb, 0)
            plsc.subcore_barrier()

            prefetch(0, 0)

            def pair(q, carry):
                process(q * 2, 0)
                process(q * 2 + 1, 1)
                return carry
            lax.fori_loop(0, nc // 2, pair, 0)
            if nc % 2:
                process(nc - 1, 0)
            plsc.subcore_barrier()

            def ob_(i, c):
                o = (sid + i * 16) * 128
                pltpu.sync_copy(acc.at[pl.ds(o, 128)],
                                out_hbm.at[cid, p, pl.ds(o, 128)])
                return c
            lax.fori_loop(0, nbt, ob_, 0)
            plsc.subcore_barrier()

    return k(feat, src, dst, w)


def _pad_rows(x, nt):
    return jnp.pad(x, ((0, nt - x.shape[0]), (0, 0)))


def _pad_edges(ei, dump):
    e = ei.shape[1]
    epad = -(-e // _EC_QUANT) * _EC_QUANT
    src = jnp.pad(ei[0], (0, epad - e))
    dst = jnp.pad(ei[1], (0, epad - e), constant_values=dump)
    return src, dst


def kernel(x_gene, x_mesh, params, ei_gg, ei_mm, ei_gm, ei_mg,
           edge_label_index):
    p = params
    del edge_label_index

    xg = jnp.pad(x_gene, ((0, _NTG - _N_GENE), (0, 20)))
    wg = jnp.pad(p["lin_g_W"], ((0, 20), (0, 0)))
    hg, stg = _mm_stats_call(xg, wg, _N_GENE)
    g = _bn_apply_call(hg, stg, p["bn_g_g"].reshape(1, _H),
                       p["bn_g_b"].reshape(1, _H), _N_GENE)
    xm = _pad_rows(x_mesh, _NTM)
    hm, stm = _mm_stats_call(xm, p["lin_m_W"], _N_MESH)
    m = _bn_apply_call(hm, stm, p["bn_m_g"].reshape(1, _H),
                       p["bn_m_b"].reshape(1, _H), _N_MESH)

    s_gg, d_gg = _pad_edges(ei_gg, _NTG - 1)
    s_mm, d_mm = _pad_edges(ei_mm, _NTM - 1)
    s_gm, d_gm = _pad_edges(ei_gm, _NTM - 1)
    s_mg, d_mg = _pad_edges(ei_mg, _NTG - 1)

    deg_gg = _deg_call(d_gg, _NPG)
    deg_mm = _deg_call(d_mm, _NPM)
    dinv_gg = _dinv_call(deg_gg)[: _NTG, 0]
    dinv_mm = _dinv_call(deg_mm)[: _NTM, 0]
    w_gg = _gcn_w_call(dinv_gg, s_gg, d_gg)
    w_mm = _gcn_w_call(dinv_mm, s_mm, d_mm)

    for l in range(2):
        hs_g, al_g, mx_g = _h_alpha_call(
            g, p[f"gat{l}_gm_Ws"], p[f"gat{l}_gm_as"].reshape(_H, 1), True)
        al_md, mx_md = _h_alpha_call(
            m, p[f"gat{l}_gm_Wd"], p[f"gat{l}_gm_ad"].reshape(_H, 1), False)
        w_e, den = _gat_w_call(al_g[:, 0], al_md[:, 0], mx_g[0], mx_md[0],
                               s_gm, d_gm, _NPM)
        num = _seg_rows_call(hs_g, s_gm, d_gm, w_e, _NPM, 1)
        nm = _finalize_call(num, den, p[f"gat{l}_gm_b"].reshape(1, _H),
                            _NTM, True)

        hs_m, al_m, mx_m = _h_alpha_call(
            m, p[f"gat{l}_mg_Ws"], p[f"gat{l}_mg_as"].reshape(_H, 1), True)
        al_gd, mx_gd = _h_alpha_call(
            g, p[f"gat{l}_mg_Wd"], p[f"gat{l}_mg_ad"].reshape(_H, 1), False)
        w_e2, den2 = _gat_w_call(al_m[:, 0], al_gd[:, 0], mx_m[0], mx_gd[0],
                                 s_mg, d_mg, _NPG)
        num2 = _seg_rows_call(hs_m.reshape(_NTM * 2, 64), s_mg, d_mg, w_e2,
                              _NPG, 2)
        ng = _finalize_call(num2, den2, p[f"gat{l}_mg_b"].reshape(1, _H),
                            _NTG, True)
        g, m = ng, nm

    for l in range(2):
        h_g = _h_call(g, p[f"gcn{l}_gg_W"])
        num_g = _seg_rows_call(h_g.reshape(_NTG * 2, 64), s_gg, d_gg, w_gg,
                               _NPG, 2)
        ng = _finalize_call(num_g, None, p[f"gcn{l}_gg_b"].reshape(1, _H),
                            _NTG, False)
        h_m = _h_call(m, p[f"gcn{l}_mm_W"])
        num_m = _seg_rows_call(h_m, s_mm, d_mm, w_mm, _NPM, 1)
        nm = _finalize_call(num_m, None, p[f"gcn{l}_mm_b"].reshape(1, _H),
                            _NTM, False)
        g, m = ng, nm

    return (g[:_N_GENE], m[:_N_MESH])


# speedup vs baseline: 6.2210x; 1.0641x over previous
import functools

import jax
import jax.numpy as jnp
from jax import lax
from jax.experimental import pallas as pl
from jax.experimental.pallas import tpu as pltpu
from jax.experimental.pallas import tpu_sc as plsc

_N_GENE = 20000
_N_MESH = 10000
_H = 128
_NTG = 20096
_NTM = 10112
_NPG = 20224
_NPM = 10112
_EC_QUANT = 32 * 128

_MESH = plsc.VectorSubcoreMesh(core_axis_name="c", subcore_axis_name="s")
_F32 = jnp.float32
_I32 = jnp.int32


def _wid():
    return lax.axis_index("s") * 2 + lax.axis_index("c")


def _mm_stats_call(x, w, n_true):
    nt, kp = x.shape
    nb = nt // 128

    def body(x_ref, w_ref, h_ref, st_ref, acc):
        i = pl.program_id(0)

        @pl.when(i == 0)
        def _():
            acc[...] = jnp.zeros_like(acc)

        h = jnp.dot(x_ref[...], w_ref[...], preferred_element_type=_F32)
        h_ref[...] = h
        acc[0:1, :] += jnp.sum(h, axis=0, keepdims=True)
        acc[1:2, :] += jnp.sum(h * h, axis=0, keepdims=True)

        @pl.when(i == nb - 1)
        def _():
            st_ref[...] = acc[...]

    return pl.pallas_call(
        body,
        grid=(nb,),
        in_specs=[
            pl.BlockSpec((128, kp), lambda i: (i, 0)),
            pl.BlockSpec((kp, _H), lambda i: (0, 0)),
        ],
        out_specs=[
            pl.BlockSpec((128, _H), lambda i: (i, 0)),
            pl.BlockSpec((8, _H), lambda i: (0, 0)),
        ],
        out_shape=[
            jax.ShapeDtypeStruct((nt, _H), _F32),
            jax.ShapeDtypeStruct((8, _H), _F32),
        ],
        scratch_shapes=[pltpu.VMEM((8, _H), _F32)],
    )(x, w)


def _bn_apply_call(h, stats, gamma, beta, n_true):
    nt = h.shape[0]
    nb = nt // 128
    inv_n = 1.0 / float(n_true)

    def body(h_ref, st_ref, g_ref, b_ref, y_ref):
        mu = st_ref[0:1, :] * inv_n
        var = st_ref[1:2, :] * inv_n - mu * mu
        scale = g_ref[...] * lax.rsqrt(var + 1e-5)
        y = (h_ref[...] - mu) * scale + b_ref[...]
        y_ref[...] = jnp.maximum(y, 0.0)

    return pl.pallas_call(
        body,
        grid=(nb,),
        in_specs=[
            pl.BlockSpec((128, _H), lambda i: (i, 0)),
            pl.BlockSpec((8, _H), lambda i: (0, 0)),
            pl.BlockSpec((1, _H), lambda i: (0, 0)),
            pl.BlockSpec((1, _H), lambda i: (0, 0)),
        ],
        out_specs=pl.BlockSpec((128, _H), lambda i: (i, 0)),
        out_shape=jax.ShapeDtypeStruct((nt, _H), _F32),
    )(h, stats, gamma, beta)


def _h_alpha_call(y, w, a_col, want_h):
    nt = y.shape[0]
    nb = nt // 128

    def body(*refs):
        if want_h:
            y_ref, w_ref, a_ref, h_ref, al_ref, m_ref, wv, mx = refs
        else:
            y_ref, w_ref, a_ref, al_ref, m_ref, wv, mx = refs
        i = pl.program_id(0)

        @pl.when(i == 0)
        def _():
            wv[...] = jnp.dot(w_ref[...], a_ref[...],
                              preferred_element_type=_F32)
            mx[0, 0] = -3.0e38

        yb = y_ref[...]
        if want_h:
            h_ref[...] = jnp.dot(yb, w_ref[...], preferred_element_type=_F32)
        av = jnp.dot(yb, wv[...], preferred_element_type=_F32)
        al_ref[...] = av
        mx[0, 0] = jnp.maximum(mx[0, 0], jnp.max(av))
        m_ref[...] = jnp.full((1, _H), mx[0, 0], _F32)

    out_specs = [
        pl.BlockSpec((128, 1), lambda i: (i, 0)),
        pl.BlockSpec((1, _H), lambda i: (0, 0)),
    ]
    out_shape = [
        jax.ShapeDtypeStruct((nt, 1), _F32),
        jax.ShapeDtypeStruct((1, _H), _F32),
    ]
    if want_h:
        out_specs.insert(0, pl.BlockSpec((128, _H), lambda i: (i, 0)))
        out_shape.insert(0, jax.ShapeDtypeStruct((nt, _H), _F32))

    return pl.pallas_call(
        body,
        grid=(nb,),
        in_specs=[
            pl.BlockSpec((128, _H), lambda i: (i, 0)),
            pl.BlockSpec((_H, _H), lambda i: (0, 0)),
            pl.BlockSpec((_H, 1), lambda i: (0, 0)),
        ],
        out_specs=out_specs,
        out_shape=out_shape,
        scratch_shapes=[pltpu.VMEM((_H, 1), _F32), pltpu.SMEM((1, 1), _F32)],
    )(y, w, a_col)


def _h_call(y, w, rs=None):
    nt = y.shape[0]
    nb = nt // 128

    def body(*refs):
        if rs is None:
            y_ref, w_ref, h_ref = refs
            yb = y_ref[...]
        else:
            y_ref, w_ref, r_ref, h_ref = refs
            yb = y_ref[...] * r_ref[...]
        h_ref[...] = jnp.dot(yb, w_ref[...], preferred_element_type=_F32)

    in_specs = [
        pl.BlockSpec((128, _H), lambda i: (i, 0)),
        pl.BlockSpec((_H, _H), lambda i: (0, 0)),
    ]
    args = [y, w]
    if rs is not None:
        in_specs.append(pl.BlockSpec((128, 1), lambda i: (i, 0)))
        args.append(rs)

    return pl.pallas_call(
        body,
        grid=(nb,),
        in_specs=in_specs,
        out_specs=pl.BlockSpec((128, _H), lambda i: (i, 0)),
        out_shape=jax.ShapeDtypeStruct((nt, _H), _F32),
    )(*args)


def _finalize_call(num, den, b, nt, gat, rs=None):
    _, p_cnt, npad, wd = num.shape
    nb = nt // 128

    in_specs = []
    for sc in range(2):
        for p in range(p_cnt):
            in_specs.append(pl.BlockSpec(
                (1, 1, 128, wd),
                functools.partial(lambda i, _sc=sc, _p=p: (_sc, _p, i, 0))))
    args = [num] * (2 * p_cnt)
    if gat:
        in_specs.append(pl.BlockSpec((32, 128), lambda i: (0, i)))
        args.append(den)
    if rs is not None:
        in_specs.append(pl.BlockSpec((128, 1), lambda i: (i, 0)))
        args.append(rs)
    in_specs.append(pl.BlockSpec((1, _H), lambda i: (0, 0)))
    args.append(b)

    def body(*refs):
        refs = list(refs)
        y_ref = refs.pop()
        b_ref = refs.pop()
        if rs is not None:
            rs_ref = refs.pop()
        if gat:
            den_ref = refs.pop()
        parts = [jnp.reshape(r[...], (128, wd)) for r in refs]
        if p_cnt == 2:
            big = jnp.concatenate([parts[0] + parts[2], parts[1] + parts[3]],
                                  axis=1)
        else:
            big = parts[0] + parts[1]
        if gat:
            dcol = lax.dot_general(den_ref[...], jnp.ones((32, 1), _F32),
                                   (((0,), (0,)), ((), ())),
                                   preferred_element_type=_F32)
            big = big / (dcol + 1e-16)
        if rs is not None:
            big = big * rs_ref[...]
        y_ref[...] = jnp.maximum(big + b_ref[...], 0.0)

    return pl.pallas_call(
        body,
        grid=(nb,),
        in_specs=in_specs,
        out_specs=pl.BlockSpec((128, _H), lambda i: (i, 0)),
        out_shape=jax.ShapeDtypeStruct((nt, _H), _F32),
    )(*args)


def _dinv_call(deg_parts):
    npad = deg_parts.shape[1]
    nb = npad // 128

    def body(d_ref, o_ref):
        deg = lax.dot_general(d_ref[...], jnp.ones((32, 1), _F32),
                              (((0,), (0,)), ((), ())),
                              preferred_element_type=_F32)
        o_ref[...] = jnp.where(deg > 0.0, lax.rsqrt(jnp.maximum(deg, 1e-12)),
                               0.0)

    return pl.pallas_call(
        body,
        grid=(nb,),
        in_specs=[pl.BlockSpec((32, 128), lambda i: (0, i))],
        out_specs=pl.BlockSpec((128, 1), lambda i: (i, 0)),
        out_shape=jax.ShapeDtypeStruct((npad, 1), _F32),
    )(deg_parts)


def _zero_1d(ref, nwords):
    z = jnp.zeros((16,), _F32)

    def bd(i, c):
        ref[pl.ds(i * 16, 16)] = z
        return c
    lax.fori_loop(0, nwords // 16, bd, 0)


def _deg_call(dst, npad):
    epad = dst.shape[0]
    ec = epad // 32

    @functools.partial(
        pl.kernel, mesh=_MESH,
        compiler_params=pltpu.CompilerParams(needs_layout_passes=False, use_tc_tiling_on_sc=False),
        out_type=jax.ShapeDtypeStruct((32, npad), _F32),
        scratch_types=[pltpu.VMEM((npad,), _F32), pltpu.VMEM((128,), _I32)],
    )
    def k(dst_hbm, out_hbm, deg_v, idx_v):
        wid = _wid()
        base = wid * ec
        _zero_1d(deg_v, npad)
        ones16 = jnp.ones((16,), _F32)

        def chunk(c, carry):
            pltpu.sync_copy(dst_hbm.at[pl.ds(base + c * 128, 128)], idx_v)
            for j in range(8):
                d16 = idx_v[pl.ds(j * 16, 16)]
                plsc.addupdate_scatter(deg_v, [d16], ones16)
            return carry
        lax.fori_loop(0, ec // 128, chunk, 0)
        pltpu.sync_copy(deg_v, out_hbm.at[wid])

    return k(dst)


def _gat_w_call(asrc, adst, msrc, mdst, src, dst, npad):
    epad = src.shape[0]
    ec = epad // 32
    ns = asrc.shape[0]
    nd = adst.shape[0]

    @functools.partial(
        pl.kernel, mesh=_MESH,
        compiler_params=pltpu.CompilerParams(needs_layout_passes=False, use_tc_tiling_on_sc=False),
        out_type=[jax.ShapeDtypeStruct((epad,), _F32),
                  jax.ShapeDtypeStruct((32, npad), _F32)],
        scratch_types=[
            pltpu.VMEM((ns,), _F32), pltpu.VMEM((nd,), _F32),
            pltpu.VMEM((npad,), _F32),
            pltpu.VMEM((128,), _I32), pltpu.VMEM((128,), _I32),
            pltpu.VMEM((128,), _F32),
            pltpu.VMEM((128,), _F32), pltpu.VMEM((128,), _F32),
        ],
    )
    def k(asrc_hbm, adst_hbm, msrc_hbm, mdst_hbm, src_hbm, dst_hbm,
          w_hbm, den_hbm, as_v, ad_v, den_v, si_v, di_v, w_v, ms_v, md_v):
        wid = _wid()
        base = wid * ec
        pltpu.sync_copy(asrc_hbm, as_v)
        pltpu.sync_copy(adst_hbm, ad_v)
        pltpu.sync_copy(msrc_hbm, ms_v)
        pltpu.sync_copy(mdst_hbm, md_v)
        _zero_1d(den_v, npad)
        msum = ms_v[pl.ds(0, 16)] + md_v[pl.ds(0, 16)]
        mb = jnp.where(msum > 0.0, msum, 0.2 * msum)

        def chunk(c, carry):
            off = base + c * 128
            pltpu.sync_copy(src_hbm.at[pl.ds(off, 128)], si_v)
            pltpu.sync_copy(dst_hbm.at[pl.ds(off, 128)], di_v)
            for j in range(8):
                s16 = si_v[pl.ds(j * 16, 16)]
                d16 = di_v[pl.ds(j * 16, 16)]
                e = plsc.load_gather(as_v, [s16]) + plsc.load_gather(ad_v, [d16])
                e = jnp.where(e > 0.0, e, 0.2 * e)
                ex = jnp.exp(e - mb)
                w_v[pl.ds(j * 16, 16)] = ex
                plsc.addupdate_scatter(den_v, [d16], ex)
            pltpu.sync_copy(w_v, w_hbm.at[pl.ds(off, 128)])
            return carry
        lax.fori_loop(0, ec // 128, chunk, 0)
        pltpu.sync_copy(den_v, den_hbm.at[wid])

    return k(asrc, adst, msrc, mdst, src, dst)


def _gcn_w_call(dinv, src, dst):
    epad = src.shape[0]
    ec = epad // 32
    nn = dinv.shape[0]

    @functools.partial(
        pl.kernel, mesh=_MESH,
        compiler_params=pltpu.CompilerParams(needs_layout_passes=False, use_tc_tiling_on_sc=False),
        out_type=jax.ShapeDtypeStruct((epad,), _F32),
        scratch_types=[
            pltpu.VMEM((nn,), _F32),
            pltpu.VMEM((128,), _I32), pltpu.VMEM((128,), _I32),
            pltpu.VMEM((128,), _F32),
        ],
    )
    def k(dinv_hbm, src_hbm, dst_hbm, w_hbm, dv, si_v, di_v, w_v):
        wid = _wid()
        base = wid * ec
        pltpu.sync_copy(dinv_hbm, dv)

        def chunk(c, carry):
            off = base + c * 128
            pltpu.sync_copy(src_hbm.at[pl.ds(off, 128)], si_v)
            pltpu.sync_copy(dst_hbm.at[pl.ds(off, 128)], di_v)
            for j in range(8):
                s16 = si_v[pl.ds(j * 16, 16)]
                d16 = di_v[pl.ds(j * 16, 16)]
                w_v[pl.ds(j * 16, 16)] = (plsc.load_gather(dv, [s16]) *
                                          plsc.load_gather(dv, [d16]))
            pltpu.sync_copy(w_v, w_hbm.at[pl.ds(off, 128)])
            return carry
        lax.fori_loop(0, ec // 128, chunk, 0)

    return k(dinv, src, dst)


def _seg_rows_call(feat, src, dst, w, npad, passes):
    has_w = w is not None
    epad = src.shape[0]
    ec = epad // 32
    nc = ec // 128
    wd = feat.shape[1]
    nvpr = wd // 16

    @functools.partial(
        pl.kernel, mesh=_MESH,
        compiler_params=pltpu.CompilerParams(needs_layout_passes=False,
                                             use_tc_tiling_on_sc=False),
        out_type=jax.ShapeDtypeStruct((2, passes, npad, wd), _F32),
        scratch_types=[
            pltpu.VMEM_SHARED((npad, wd), _F32),
            pltpu.VMEM((2, 128), _I32), pltpu.VMEM((2, 128), _I32),
            pltpu.VMEM((128,), _I32), pltpu.VMEM((128,), _F32),
            pltpu.VMEM((2, 128, wd), _F32), pltpu.VMEM((128, wd), _F32),
            pltpu.SemaphoreType.DMA, pltpu.SemaphoreType.DMA,
        ],
    )
    def k(feat_hbm, src_hbm, dst_hbm, *rest):
        if has_w:
            (w_hbm, out_hbm, acc, si2, gi2, di_v, w_v, rows2, zrow_v,
             gsem0, gsem1) = rest
        else:
            (out_hbm, acc, si2, gi2, di_v, w_v, rows2, zrow_v,
             gsem0, gsem1) = rest
        gsem = (gsem0, gsem1)
        cid = lax.axis_index("c")
        sid = lax.axis_index("s")
        wid = sid * 2 + cid
        base = wid * ec
        z16 = jnp.zeros((16,), _F32)

        def zr(i, c):
            for j in range(nvpr):
                zrow_v[i, pl.ds(j * 16, 16)] = z16
            return c
        lax.fori_loop(0, 128, zr, 0)

        nblk = npad // 128
        nbt = (nblk - sid + 15) // 16
        for p in range(passes):

            def prefetch(c, bk):
                off = base + c * 128
                pltpu.sync_copy(src_hbm.at[pl.ds(off, 128)], si2.at[bk])
                if passes == 1:
                    pltpu.async_copy(feat_hbm.at[si2.at[bk]], rows2.at[bk],
                                     gsem[bk])
                else:
                    for j in range(8):
                        s16 = si2[bk, pl.ds(j * 16, 16)]
                        gi2[bk, pl.ds(j * 16, 16)] = s16 * passes + p
                    pltpu.async_copy(feat_hbm.at[gi2.at[bk]], rows2.at[bk],
                                     gsem[bk])

            def process(c, bk):
                off = base + c * 128
                pltpu.sync_copy(dst_hbm.at[pl.ds(off, 128)], di_v)
                if has_w:
                    pltpu.sync_copy(w_hbm.at[pl.ds(off, 128)], w_v)
                pltpu.make_async_copy(feat_hbm.at[pl.ds(0, 128)],
                                      rows2.at[bk], gsem[bk]).wait()

                @pl.when(c + 1 < nc)
                def _():
                    prefetch(c + 1, 1 - bk)

                if has_w:
                    def scale(i, cc):
                        for rr in range(4):
                            r = i * 4 + rr
                            wv = plsc.load_gather(w_v,
                                                  [jnp.full((16,), r, _I32)])
                            for j in range(nvpr):
                                rows2[bk, r, pl.ds(j * 16, 16)] = (
                                    rows2[bk, r, pl.ds(j * 16, 16)] * wv)
                        return cc
                    lax.fori_loop(0, 32, scale, 0)
                pltpu.sync_copy(rows2.at[bk], acc.at[di_v], add=True)

            def zb(i, c):
                pltpu.sync_copy(zrow_v, acc.at[pl.ds((sid + i * 16) * 128,
                                                     128)])
                return c
            lax.fori_loop(0, nbt, zb, 0)
            plsc.subcore_barrier()

            prefetch(0, 0)

            def pair(q, carry):
                process(q * 2, 0)
                process(q * 2 + 1, 1)
                return carry
            lax.fori_loop(0, nc // 2, pair, 0)
            if nc % 2:
                process(nc - 1, 0)
            plsc.subcore_barrier()

            def ob_(i, c):
                o = (sid + i * 16) * 128
                pltpu.sync_copy(acc.at[pl.ds(o, 128)],
                                out_hbm.at[cid, p, pl.ds(o, 128)])
                return c
            lax.fori_loop(0, nbt, ob_, 0)
            plsc.subcore_barrier()

    if has_w:
        return k(feat, src, dst, w)
    return k(feat, src, dst)


def _pad_rows(x, nt):
    return jnp.pad(x, ((0, nt - x.shape[0]), (0, 0)))


def _pad_edges(ei, dump):
    e = ei.shape[1]
    epad = -(-e // _EC_QUANT) * _EC_QUANT
    src = jnp.pad(ei[0], (0, epad - e))
    dst = jnp.pad(ei[1], (0, epad - e), constant_values=dump)
    return src, dst


def kernel(x_gene, x_mesh, params, ei_gg, ei_mm, ei_gm, ei_mg,
           edge_label_index):
    p = params
    del edge_label_index

    xg = jnp.pad(x_gene, ((0, _NTG - _N_GENE), (0, 20)))
    wg = jnp.pad(p["lin_g_W"], ((0, 20), (0, 0)))
    hg, stg = _mm_stats_call(xg, wg, _N_GENE)
    g = _bn_apply_call(hg, stg, p["bn_g_g"].reshape(1, _H),
                       p["bn_g_b"].reshape(1, _H), _N_GENE)
    xm = _pad_rows(x_mesh, _NTM)
    hm, stm = _mm_stats_call(xm, p["lin_m_W"], _N_MESH)
    m = _bn_apply_call(hm, stm, p["bn_m_g"].reshape(1, _H),
                       p["bn_m_b"].reshape(1, _H), _N_MESH)

    s_gg, d_gg = _pad_edges(ei_gg, _NTG - 1)
    s_mm, d_mm = _pad_edges(ei_mm, _NTM - 1)
    s_gm, d_gm = _pad_edges(ei_gm, _NTM - 1)
    s_mg, d_mg = _pad_edges(ei_mg, _NTG - 1)

    deg_gg = _deg_call(d_gg, _NPG)
    deg_mm = _deg_call(d_mm, _NPM)
    dinv_gg = _dinv_call(deg_gg)
    dinv_mm = _dinv_call(deg_mm)

    for l in range(2):
        hs_g, al_g, mx_g = _h_alpha_call(
            g, p[f"gat{l}_gm_Ws"], p[f"gat{l}_gm_as"].reshape(_H, 1), True)
        al_md, mx_md = _h_alpha_call(
            m, p[f"gat{l}_gm_Wd"], p[f"gat{l}_gm_ad"].reshape(_H, 1), False)
        w_e, den = _gat_w_call(al_g[:, 0], al_md[:, 0], mx_g[0], mx_md[0],
                               s_gm, d_gm, _NPM)
        num = _seg_rows_call(hs_g, s_gm, d_gm, w_e, _NPM, 1)
        nm = _finalize_call(num, den, p[f"gat{l}_gm_b"].reshape(1, _H),
                            _NTM, True)

        hs_m, al_m, mx_m = _h_alpha_call(
            m, p[f"gat{l}_mg_Ws"], p[f"gat{l}_mg_as"].reshape(_H, 1), True)
        al_gd, mx_gd = _h_alpha_call(
            g, p[f"gat{l}_mg_Wd"], p[f"gat{l}_mg_ad"].reshape(_H, 1), False)
        w_e2, den2 = _gat_w_call(al_m[:, 0], al_gd[:, 0], mx_m[0], mx_gd[0],
                                 s_mg, d_mg, _NPG)
        num2 = _seg_rows_call(hs_m.reshape(_NTM * 2, 64), s_mg, d_mg, w_e2,
                              _NPG, 2)
        ng = _finalize_call(num2, den2, p[f"gat{l}_mg_b"].reshape(1, _H),
                            _NTG, True)
        g, m = ng, nm

    for l in range(2):
        h_g = _h_call(g, p[f"gcn{l}_gg_W"], dinv_gg[: _NTG])
        num_g = _seg_rows_call(h_g.reshape(_NTG * 2, 64), s_gg, d_gg, None,
                               _NPG, 2)
        ng = _finalize_call(num_g, None, p[f"gcn{l}_gg_b"].reshape(1, _H),
                            _NTG, False, dinv_gg)
        h_m = _h_call(m, p[f"gcn{l}_mm_W"], dinv_mm[: _NTM])
        num_m = _seg_rows_call(h_m, s_mm, d_mm, None, _NPM, 1)
        nm = _finalize_call(num_m, None, p[f"gcn{l}_mm_b"].reshape(1, _H),
                            _NTM, False, dinv_mm)
        g, m = ng, nm

    return (g[:_N_GENE], m[:_N_MESH])
